# baseline (speedup 1.0000x reference)
import functools

import jax
import jax.numpy as jnp
from jax import lax
from jax.experimental import pallas as pl
from jax.experimental.pallas import tpu as pltpu
from jax.experimental.pallas import tpu_sc as plsc

NUM_TYPES = 100
N_SAMPLES = 3_200_000
N_PROPS = 16

NW = 32
NBLK = 25
CHUNK = NBLK * 128
NBLOCKS = N_SAMPLES // 128

SC_BLOCKS = 12600
NCHUNKS = SC_BLOCKS // NBLK
SLOTS = -(-NCHUNKS // NW)
TC_G = 40
TC_OFF = SC_BLOCKS // TC_G
TC_STEPS = (NBLOCKS - SC_BLOCKS) // TC_G


def _sc_partials(data4, ids):
  mesh = plsc.VectorSubcoreMesh(core_axis_name="c", subcore_axis_name="s")

  @functools.partial(
      pl.kernel,
      out_type=jax.ShapeDtypeStruct((NW * 2 * NUM_TYPES * N_PROPS,),
                                    jnp.float32),
      mesh=mesh,
      compiler_params=pltpu.CompilerParams(
          needs_layout_passes=False, use_tc_tiling_on_sc=False),
      scratch_types=[
          pltpu.VMEM((NBLK, 8, 128), jnp.float32),
          pltpu.VMEM((NBLK, 8, 128), jnp.float32),
          pltpu.VMEM((NBLK, 8, 128), jnp.float32),
          pltpu.VMEM((NBLK, 8, 128), jnp.float32),
          pltpu.VMEM((16,), jnp.int32),
          pltpu.VMEM((16,), jnp.int32),
          pltpu.VMEM((16,), jnp.int32),
          pltpu.VMEM((16,), jnp.int32),
          pltpu.VMEM((CHUNK,), jnp.int32),
          pltpu.VMEM((NUM_TYPES * N_PROPS,), jnp.float32),
          pltpu.VMEM((NUM_TYPES * N_PROPS,), jnp.float32),
          pltpu.VMEM((256,), jnp.float32),
          pltpu.SemaphoreType.DMA,
          pltpu.SemaphoreType.DMA,
          pltpu.SemaphoreType.DMA,
          pltpu.SemaphoreType.DMA,
      ],
  )
  def k(data_hbm, ids_hbm, out_hbm, lo0, hi0, lo1, hi1, idsF0, idsL0,
        idsF1, idsL1, idsfull, acc, cnt, tt, sd0, sd1, si0, si1):
    wid = lax.axis_index("c") * 16 + lax.axis_index("s")
    nc = jnp.where(wid < NCHUNKS - (SLOTS - 1) * NW, SLOTS, SLOTS - 1)
    iota16 = lax.iota(jnp.int32, 16)
    iota16x16 = iota16 * 16
    zeros16 = jnp.zeros((16,), jnp.float32)
    ones16 = jnp.ones((16,), jnp.float32)

    def zbody(kk, _):
      acc[pl.ds(kk * 16, 16)] = zeros16
      cnt[pl.ds(kk * 16, 16)] = zeros16
      return 0
    lax.fori_loop(0, NUM_TYPES, zbody, 0)

    def chunk_of(slot):
      return wid + jnp.minimum(slot, nc - 1) * NW

    def start(slot, lo, hi, idsF, idsL, sd, si):
      c = chunk_of(slot)
      b = c * NBLK
      pltpu.make_async_copy(data_hbm.at[0, pl.ds(b, NBLK)], lo, sd).start()
      pltpu.make_async_copy(data_hbm.at[1, pl.ds(b, NBLK)], hi, sd).start()
      pltpu.make_async_copy(ids_hbm.at[pl.ds(c * CHUNK, 16)], idsF,
                            si).start()
      pltpu.make_async_copy(ids_hbm.at[pl.ds(c * CHUNK + CHUNK - 16, 16)],
                            idsL, si).start()

    def wait(slot, lo, hi, idsF, idsL, sd, si):
      c = chunk_of(slot)
      b = c * NBLK
      pltpu.make_async_copy(data_hbm.at[0, pl.ds(b, NBLK)], lo, sd).wait()
      pltpu.make_async_copy(data_hbm.at[1, pl.ds(b, NBLK)], hi, sd).wait()
      pltpu.make_async_copy(ids_hbm.at[pl.ds(c * CHUNK, 16)], idsF,
                            si).wait()
      pltpu.make_async_copy(ids_hbm.at[pl.ds(c * CHUNK + CHUNK - 16, 16)],
                            idsL, si).wait()

    def lanesum_from_tt():
      tot = plsc.load_gather(tt, [iota16x16])
      for l in range(1, 16):
        tot = tot + plsc.load_gather(tt, [iota16x16 + l])
      return tot

    def flush_accp(accp, seg, n_samples):
      for p in range(16):
        tt[pl.ds(p * 16, 16)] = accp[p]
      tot = lanesum_from_tt()
      idx = jnp.full((16,), seg * 16, jnp.int32) + iota16
      plsc.addupdate_scatter(acc, [idx], tot)
      plsc.addupdate_scatter(cnt, [idx],
                             jnp.full((16,), n_samples, jnp.float32))

    def accum_block(lo, hi, blk, accp):
      out = list(accp)
      for half, buf in ((0, lo), (1, hi)):
        for j in range(8):
          p = half * 8 + j
          a = out[p]
          for kk in range(8):
            v = buf[blk, j, pl.ds(kk * 16, 16)]
            a = a + v * v
          out[p] = a
      return tuple(out)

    def process(slot, lo, hi, idsF, idsL):
      first = idsF[...][0]
      last = idsL[...][15]
      uniform = first == last

      @pl.when(uniform)
      def _fast():
        accp = lax.fori_loop(
            0, NBLK, lambda blk, accs: accum_block(lo, hi, blk, accs),
            tuple(zeros16 for _ in range(16)))
        flush_accp(accp, first, float(CHUNK))

      @pl.when(jnp.logical_not(uniform))
      def _slow():
        c = chunk_of(slot)
        pltpu.sync_copy(ids_hbm.at[pl.ds(c * CHUNK, CHUNK)], idsfull)

        def blk_body(blk, _):
          boff = blk * 128
          bfirst = idsfull[pl.ds(boff, 16)][0]
          blast = idsfull[pl.ds(boff + 112, 16)][15]

          @pl.when(bfirst == blast)
          def _ublock():
            accp = accum_block(lo, hi, blk, tuple(zeros16 for _ in range(16)))
            flush_accp(accp, bfirst, 128.0)

          @pl.when(jnp.logical_not(bfirst == blast))
          def _bblock():
            for kk in range(8):
              segs = idsfull[pl.ds(boff + kk * 16, 16)]
              for half, buf in ((0, lo), (1, hi)):
                for j in range(8):
                  tt[pl.ds((half * 8 + j) * 16, 16)] = (
                      buf[blk, j, pl.ds(kk * 16, 16)])
              for l in range(16):
                col = plsc.load_gather(tt, [iota16x16 + l])
                idx = jnp.full((16,), segs[l] * 16, jnp.int32) + iota16
                plsc.addupdate_scatter(acc, [idx], col * col)
                plsc.addupdate_scatter(cnt, [idx], ones16)
          return 0
        lax.fori_loop(0, NBLK, blk_body, 0)

    start(0, lo0, hi0, idsF0, idsL0, sd0, si0)
    start(1, lo1, hi1, idsF1, idsL1, sd1, si1)

    def outer(kk, _):
      n0 = 2 * kk

      @pl.when(n0 < nc)
      def _w0():
        wait(n0, lo0, hi0, idsF0, idsL0, sd0, si0)
        process(n0, lo0, hi0, idsF0, idsL0)

      @pl.when(n0 + 2 < nc)
      def _s0():
        start(n0 + 2, lo0, hi0, idsF0, idsL0, sd0, si0)

      @pl.when(n0 + 1 < nc)
      def _w1():
        wait(n0 + 1, lo1, hi1, idsF1, idsL1, sd1, si1)
        process(n0 + 1, lo1, hi1, idsF1, idsL1)

      @pl.when(n0 + 3 < nc)
      def _s1():
        start(n0 + 3, lo1, hi1, idsF1, idsL1, sd1, si1)
      return 0
    lax.fori_loop(0, (SLOTS + 1) // 2, outer, 0)

    base = wid * 2 * NUM_TYPES * N_PROPS
    pltpu.sync_copy(acc, out_hbm.at[pl.ds(base, NUM_TYPES * N_PROPS)])
    pltpu.sync_copy(
        cnt, out_hbm.at[pl.ds(base + NUM_TYPES * N_PROPS,
                              NUM_TYPES * N_PROPS)])

  return k(data4, ids)


def _tc_main(data4, ids2d):
  def body(lo_ref, hi_ref, ids_ref, y2_ref, cnt_ref,
           acc0, acc1, y2acc, cntacc, rseg, rn):
    i = pl.program_id(0)
    ids_blk = ids_ref[...]
    cmin = jnp.min(ids_blk)
    cmax = jnp.max(ids_blk)
    uniform = cmin == cmax
    z8 = jnp.zeros((8, 128), jnp.float32)

    @pl.when(i == 0)
    def _init():
      y2acc[...] = jnp.zeros((NUM_TYPES, N_PROPS), jnp.float32)
      cntacc[...] = jnp.zeros((NUM_TYPES, N_PROPS), jnp.float32)
      acc0[...] = z8
      acc1[...] = z8
      rseg[0] = cmin
      rn[0] = 0.0

    def flush():
      rs = rseg[0]
      row = jnp.concatenate(
          [jnp.sum(acc0[...], axis=1), jnp.sum(acc1[...], axis=1)])[None, :]
      y2acc[pl.ds(rs, 1), :] = y2acc[pl.ds(rs, 1), :] + row
      cntacc[pl.ds(rs, 1), :] = cntacc[pl.ds(rs, 1), :] + rn[0]
      acc0[...] = z8
      acc1[...] = z8
      rn[0] = 0.0

    @pl.when(uniform)
    def _u():
      @pl.when(cmin != rseg[0])
      def _sw():
        flush()
        rseg[0] = cmin
      a0 = acc0[...]
      a1 = acc1[...]
      for g in range(TC_G):
        v0 = lo_ref[0, g]
        v1 = hi_ref[0, g]
        a0 = a0 + v0 * v0
        a1 = a1 + v1 * v1
      acc0[...] = a0
      acc1[...] = a1
      rn[0] = rn[0] + float(TC_G * 128)

    @pl.when(jnp.logical_not(uniform))
    def _b():
      flush()
      iota100 = lax.broadcasted_iota(jnp.int32, (NUM_TYPES, 1), 0)
      for g in range(TC_G):
        bid = ids_blk[pl.ds(g, 1), :] if False else ids_blk[g:g + 1, :]
        bmin = jnp.min(bid)
        bmax = jnp.max(bid)
        v0 = lo_ref[0, g]
        v1 = hi_ref[0, g]
        sq0 = v0 * v0
        sq1 = v1 * v1

        @pl.when(bmin == bmax)
        def _ub(sq0=sq0, sq1=sq1, bmin=bmin):
          row = jnp.concatenate(
              [jnp.sum(sq0, axis=1), jnp.sum(sq1, axis=1)])[None, :]
          y2acc[pl.ds(bmin, 1), :] = y2acc[pl.ds(bmin, 1), :] + row
          cntacc[pl.ds(bmin, 1), :] = cntacc[pl.ds(bmin, 1), :] + 128.0

        @pl.when(bmin != bmax)
        def _bb(sq0=sq0, sq1=sq1, bid=bid):
          oh = (iota100 == bid).astype(jnp.float32)
          dn = (((1,), (1,)), ((), ()))
          p0 = lax.dot_general(oh, sq0, dn,
                               preferred_element_type=jnp.float32)
          p1 = lax.dot_general(oh, sq1, dn,
                               preferred_element_type=jnp.float32)
          y2acc[...] = y2acc[...] + jnp.concatenate([p0, p1], axis=1)
          cntacc[...] = cntacc[...] + jnp.sum(oh, axis=1, keepdims=True)
      rseg[0] = cmax

    @pl.when(i == TC_STEPS - 1)
    def _fin():
      flush()
      y2_ref[...] = y2acc[...]
      cnt_ref[...] = cntacc[...]

  return pl.pallas_call(
      body,
      grid=(TC_STEPS,),
      in_specs=[
          pl.BlockSpec((1, TC_G, 8, 128), lambda i: (0, TC_OFF + i, 0, 0)),
          pl.BlockSpec((1, TC_G, 8, 128), lambda i: (1, TC_OFF + i, 0, 0)),
          pl.BlockSpec((TC_G, 128), lambda i: (TC_OFF + i, 0)),
      ],
      out_specs=[
          pl.BlockSpec((NUM_TYPES, N_PROPS), lambda i: (0, 0)),
          pl.BlockSpec((NUM_TYPES, N_PROPS), lambda i: (0, 0)),
      ],
      out_shape=[
          jax.ShapeDtypeStruct((NUM_TYPES, N_PROPS), jnp.float32),
          jax.ShapeDtypeStruct((NUM_TYPES, N_PROPS), jnp.float32),
      ],
      scratch_shapes=[
          pltpu.VMEM((8, 128), jnp.float32),
          pltpu.VMEM((8, 128), jnp.float32),
          pltpu.VMEM((NUM_TYPES, N_PROPS), jnp.float32),
          pltpu.VMEM((NUM_TYPES, N_PROPS), jnp.float32),
          pltpu.SMEM((1,), jnp.int32),
          pltpu.SMEM((1,), jnp.float32),
      ],
  )(data4, data4, ids2d)


def _tc_finalize(parts, y2tc, cnttc):
  d = NUM_TYPES * N_PROPS

  def body(p_ref, ytc_ref, ctc_ref, o_ref):
    y2 = ytc_ref[...]
    c = ctc_ref[...]
    for w in range(NW):
      y2 = y2 + p_ref[pl.ds(w * 2 * d, d)]
      c = c + p_ref[pl.ds(w * 2 * d + d, d)]
    o_ref[...] = jnp.where(c > 0.0, jnp.sqrt(y2 / jnp.maximum(c, 1.0)),
                           jnp.float32(1.0))

  return pl.pallas_call(
      body,
      out_shape=jax.ShapeDtypeStruct((d,), jnp.float32),
  )(parts, y2tc, cnttc)


@jax.jit
def kernel(data, segment_ids):
  ids = segment_ids.astype(jnp.int32)
  data4 = data.T.reshape(2, 8, N_SAMPLES // 128, 128).transpose(0, 2, 1, 3)
  ids2d = ids.reshape(NBLOCKS, 128)
  parts = _sc_partials(data4, ids)
  y2tc, cnttc = _tc_main(data4, ids2d)
  return _tc_finalize(parts, y2tc.reshape(-1),
                      cnttc.reshape(-1)).reshape(NUM_TYPES, N_PROPS)


# speedup vs baseline: 1.3047x; 1.3047x over previous
import functools

import jax
import jax.numpy as jnp
from jax import lax
from jax.experimental import pallas as pl
from jax.experimental.pallas import tpu as pltpu
from jax.experimental.pallas import tpu_sc as plsc

NUM_TYPES = 100
N_SAMPLES = 3_200_000
N_PROPS = 16

NW = 32
NBLK = 25
CHUNK = NBLK * 128
NBLOCKS = N_SAMPLES // 128

SC_BLOCKS = 19400
NCHUNKS = SC_BLOCKS // NBLK
SLOTS = -(-NCHUNKS // NW)
TC_G = 80
TC_OFF = SC_BLOCKS // TC_G
TC_STEPS = (NBLOCKS - SC_BLOCKS) // TC_G


def _sc_partials(data4, ids):
  mesh = plsc.VectorSubcoreMesh(core_axis_name="c", subcore_axis_name="s")

  @functools.partial(
      pl.kernel,
      out_type=jax.ShapeDtypeStruct((NW * 2 * NUM_TYPES * N_PROPS,),
                                    jnp.float32),
      mesh=mesh,
      compiler_params=pltpu.CompilerParams(
          needs_layout_passes=False, use_tc_tiling_on_sc=False),
      scratch_types=[
          pltpu.VMEM((NBLK, 8, 128), jnp.float32),
          pltpu.VMEM((NBLK, 8, 128), jnp.float32),
          pltpu.VMEM((NBLK, 8, 128), jnp.float32),
          pltpu.VMEM((NBLK, 8, 128), jnp.float32),
          pltpu.VMEM((16,), jnp.int32),
          pltpu.VMEM((16,), jnp.int32),
          pltpu.VMEM((16,), jnp.int32),
          pltpu.VMEM((16,), jnp.int32),
          pltpu.VMEM((CHUNK,), jnp.int32),
          pltpu.VMEM((NUM_TYPES * N_PROPS,), jnp.float32),
          pltpu.VMEM((NUM_TYPES * N_PROPS,), jnp.float32),
          pltpu.VMEM((256,), jnp.float32),
          pltpu.SemaphoreType.DMA,
          pltpu.SemaphoreType.DMA,
          pltpu.SemaphoreType.DMA,
          pltpu.SemaphoreType.DMA,
      ],
  )
  def k(data_hbm, ids_hbm, out_hbm, lo0, hi0, lo1, hi1, idsF0, idsL0,
        idsF1, idsL1, idsfull, acc, cnt, tt, sd0, sd1, si0, si1):
    wid = lax.axis_index("c") * 16 + lax.axis_index("s")
    nc = jnp.where(wid < NCHUNKS - (SLOTS - 1) * NW, SLOTS, SLOTS - 1)
    iota16 = lax.iota(jnp.int32, 16)
    iota16x16 = iota16 * 16
    zeros16 = jnp.zeros((16,), jnp.float32)
    ones16 = jnp.ones((16,), jnp.float32)

    def zbody(kk, _):
      acc[pl.ds(kk * 16, 16)] = zeros16
      cnt[pl.ds(kk * 16, 16)] = zeros16
      return 0
    lax.fori_loop(0, NUM_TYPES, zbody, 0)

    def chunk_of(slot):
      return wid + jnp.minimum(slot, nc - 1) * NW

    def start(slot, lo, hi, idsF, idsL, sd, si):
      c = chunk_of(slot)
      b = c * NBLK
      pltpu.make_async_copy(data_hbm.at[0, pl.ds(b, NBLK)], lo, sd).start()
      pltpu.make_async_copy(data_hbm.at[1, pl.ds(b, NBLK)], hi, sd).start()
      pltpu.make_async_copy(ids_hbm.at[pl.ds(c * CHUNK, 16)], idsF,
                            si).start()
      pltpu.make_async_copy(ids_hbm.at[pl.ds(c * CHUNK + CHUNK - 16, 16)],
                            idsL, si).start()

    def wait(slot, lo, hi, idsF, idsL, sd, si):
      c = chunk_of(slot)
      b = c * NBLK
      pltpu.make_async_copy(data_hbm.at[0, pl.ds(b, NBLK)], lo, sd).wait()
      pltpu.make_async_copy(data_hbm.at[1, pl.ds(b, NBLK)], hi, sd).wait()
      pltpu.make_async_copy(ids_hbm.at[pl.ds(c * CHUNK, 16)], idsF,
                            si).wait()
      pltpu.make_async_copy(ids_hbm.at[pl.ds(c * CHUNK + CHUNK - 16, 16)],
                            idsL, si).wait()

    def lanesum_from_tt():
      tot = plsc.load_gather(tt, [iota16x16])
      for l in range(1, 16):
        tot = tot + plsc.load_gather(tt, [iota16x16 + l])
      return tot

    def flush_accp(accp, seg, n_samples):
      for p in range(16):
        tt[pl.ds(p * 16, 16)] = accp[p]
      tot = lanesum_from_tt()
      idx = jnp.full((16,), seg * 16, jnp.int32) + iota16
      plsc.addupdate_scatter(acc, [idx], tot)
      plsc.addupdate_scatter(cnt, [idx],
                             jnp.full((16,), n_samples, jnp.float32))

    def accum_block(lo, hi, blk, accp):
      out = list(accp)
      for half, buf in ((0, lo), (1, hi)):
        for j in range(8):
          p = half * 8 + j
          a = out[p]
          for kk in range(8):
            v = buf[blk, j, pl.ds(kk * 16, 16)]
            a = a + v * v
          out[p] = a
      return tuple(out)

    def process(slot, lo, hi, idsF, idsL):
      first = idsF[...][0]
      last = idsL[...][15]
      uniform = first == last

      @pl.when(uniform)
      def _fast():
        accp = lax.fori_loop(
            0, NBLK, lambda blk, accs: accum_block(lo, hi, blk, accs),
            tuple(zeros16 for _ in range(16)))
        flush_accp(accp, first, float(CHUNK))

      @pl.when(jnp.logical_not(uniform))
      def _slow():
        c = chunk_of(slot)
        pltpu.sync_copy(ids_hbm.at[pl.ds(c * CHUNK, CHUNK)], idsfull)

        def blk_body(blk, _):
          boff = blk * 128
          bfirst = idsfull[pl.ds(boff, 16)][0]
          blast = idsfull[pl.ds(boff + 112, 16)][15]

          @pl.when(bfirst == blast)
          def _ublock():
            accp = accum_block(lo, hi, blk, tuple(zeros16 for _ in range(16)))
            flush_accp(accp, bfirst, 128.0)

          @pl.when(jnp.logical_not(bfirst == blast))
          def _bblock():
            for kk in range(8):
              segs = idsfull[pl.ds(boff + kk * 16, 16)]
              for half, buf in ((0, lo), (1, hi)):
                for j in range(8):
                  tt[pl.ds((half * 8 + j) * 16, 16)] = (
                      buf[blk, j, pl.ds(kk * 16, 16)])
              for l in range(16):
                col = plsc.load_gather(tt, [iota16x16 + l])
                idx = jnp.full((16,), segs[l] * 16, jnp.int32) + iota16
                plsc.addupdate_scatter(acc, [idx], col * col)
                plsc.addupdate_scatter(cnt, [idx], ones16)
          return 0
        lax.fori_loop(0, NBLK, blk_body, 0)

    start(0, lo0, hi0, idsF0, idsL0, sd0, si0)
    start(1, lo1, hi1, idsF1, idsL1, sd1, si1)

    def outer(kk, _):
      n0 = 2 * kk

      @pl.when(n0 < nc)
      def _w0():
        wait(n0, lo0, hi0, idsF0, idsL0, sd0, si0)
        process(n0, lo0, hi0, idsF0, idsL0)

      @pl.when(n0 + 2 < nc)
      def _s0():
        start(n0 + 2, lo0, hi0, idsF0, idsL0, sd0, si0)

      @pl.when(n0 + 1 < nc)
      def _w1():
        wait(n0 + 1, lo1, hi1, idsF1, idsL1, sd1, si1)
        process(n0 + 1, lo1, hi1, idsF1, idsL1)

      @pl.when(n0 + 3 < nc)
      def _s1():
        start(n0 + 3, lo1, hi1, idsF1, idsL1, sd1, si1)
      return 0
    lax.fori_loop(0, (SLOTS + 1) // 2, outer, 0)

    base = wid * 2 * NUM_TYPES * N_PROPS
    pltpu.sync_copy(acc, out_hbm.at[pl.ds(base, NUM_TYPES * N_PROPS)])
    pltpu.sync_copy(
        cnt, out_hbm.at[pl.ds(base + NUM_TYPES * N_PROPS,
                              NUM_TYPES * N_PROPS)])

  return k(data4, ids)


def _tc_main(data4, ids2d):
  def body(lo_ref, hi_ref, ids_ref, y2_ref, cnt_ref,
           acc0, acc1, y2acc, cntacc, rseg, rn):
    i = pl.program_id(0)
    ids_blk = ids_ref[...]
    cmin = jnp.min(ids_blk)
    cmax = jnp.max(ids_blk)
    uniform = cmin == cmax
    z8 = jnp.zeros((8, 128), jnp.float32)

    @pl.when(i == 0)
    def _init():
      y2acc[...] = jnp.zeros((NUM_TYPES, N_PROPS), jnp.float32)
      cntacc[...] = jnp.zeros((NUM_TYPES, N_PROPS), jnp.float32)
      acc0[...] = z8
      acc1[...] = z8
      rseg[0] = cmin
      rn[0] = 0.0

    def flush():
      rs = rseg[0]
      row = jnp.concatenate(
          [jnp.sum(acc0[...], axis=1), jnp.sum(acc1[...], axis=1)])[None, :]
      y2acc[pl.ds(rs, 1), :] = y2acc[pl.ds(rs, 1), :] + row
      cntacc[pl.ds(rs, 1), :] = cntacc[pl.ds(rs, 1), :] + rn[0]
      acc0[...] = z8
      acc1[...] = z8
      rn[0] = 0.0

    @pl.when(uniform)
    def _u():
      @pl.when(cmin != rseg[0])
      def _sw():
        flush()
        rseg[0] = cmin
      a0 = acc0[...]
      a1 = acc1[...]
      for g in range(TC_G):
        v0 = lo_ref[0, g]
        v1 = hi_ref[0, g]
        a0 = a0 + v0 * v0
        a1 = a1 + v1 * v1
      acc0[...] = a0
      acc1[...] = a1
      rn[0] = rn[0] + float(TC_G * 128)

    @pl.when(jnp.logical_not(uniform))
    def _b():
      flush()
      iota100 = lax.broadcasted_iota(jnp.int32, (NUM_TYPES, 1), 0)

      def blk_body(g, _):
        bid = ids_ref[pl.ds(g, 1), :]
        bmin = jnp.min(bid)
        bmax = jnp.max(bid)
        v0 = lo_ref[0, g]
        v1 = hi_ref[0, g]
        sq0 = v0 * v0
        sq1 = v1 * v1

        @pl.when(bmin == bmax)
        def _ub():
          row = jnp.concatenate(
              [jnp.sum(sq0, axis=1), jnp.sum(sq1, axis=1)])[None, :]
          y2acc[pl.ds(bmin, 1), :] = y2acc[pl.ds(bmin, 1), :] + row
          cntacc[pl.ds(bmin, 1), :] = cntacc[pl.ds(bmin, 1), :] + 128.0

        @pl.when(bmin != bmax)
        def _bb():
          oh = (iota100 == bid).astype(jnp.float32)
          dn = (((1,), (1,)), ((), ()))
          p0 = lax.dot_general(oh, sq0, dn,
                               preferred_element_type=jnp.float32)
          p1 = lax.dot_general(oh, sq1, dn,
                               preferred_element_type=jnp.float32)
          y2acc[...] = y2acc[...] + jnp.concatenate([p0, p1], axis=1)
          cntacc[...] = cntacc[...] + jnp.sum(oh, axis=1, keepdims=True)
        return 0

      lax.fori_loop(0, TC_G, blk_body, 0)
      rseg[0] = cmax

    @pl.when(i == TC_STEPS - 1)
    def _fin():
      flush()
      y2_ref[...] = y2acc[...]
      cnt_ref[...] = cntacc[...]

  return pl.pallas_call(
      body,
      grid=(TC_STEPS,),
      in_specs=[
          pl.BlockSpec((1, TC_G, 8, 128), lambda i: (0, TC_OFF + i, 0, 0)),
          pl.BlockSpec((1, TC_G, 8, 128), lambda i: (1, TC_OFF + i, 0, 0)),
          pl.BlockSpec((TC_G, 128), lambda i: (TC_OFF + i, 0)),
      ],
      out_specs=[
          pl.BlockSpec((NUM_TYPES, N_PROPS), lambda i: (0, 0)),
          pl.BlockSpec((NUM_TYPES, N_PROPS), lambda i: (0, 0)),
      ],
      out_shape=[
          jax.ShapeDtypeStruct((NUM_TYPES, N_PROPS), jnp.float32),
          jax.ShapeDtypeStruct((NUM_TYPES, N_PROPS), jnp.float32),
      ],
      scratch_shapes=[
          pltpu.VMEM((8, 128), jnp.float32),
          pltpu.VMEM((8, 128), jnp.float32),
          pltpu.VMEM((NUM_TYPES, N_PROPS), jnp.float32),
          pltpu.VMEM((NUM_TYPES, N_PROPS), jnp.float32),
          pltpu.SMEM((1,), jnp.int32),
          pltpu.SMEM((1,), jnp.float32),
      ],
  )(data4, data4, ids2d)


def _tc_finalize(parts, y2tc, cnttc):
  d = NUM_TYPES * N_PROPS

  def body(p_ref, ytc_ref, ctc_ref, o_ref):
    y2 = ytc_ref[...]
    c = ctc_ref[...]
    for w in range(NW):
      y2 = y2 + p_ref[pl.ds(w * 2 * d, d)]
      c = c + p_ref[pl.ds(w * 2 * d + d, d)]
    o_ref[...] = jnp.where(c > 0.0, jnp.sqrt(y2 / jnp.maximum(c, 1.0)),
                           jnp.float32(1.0))

  return pl.pallas_call(
      body,
      out_shape=jax.ShapeDtypeStruct((d,), jnp.float32),
  )(parts, y2tc, cnttc)


@jax.jit
def kernel(data, segment_ids):
  ids = segment_ids.astype(jnp.int32)
  data4 = data.T.reshape(2, 8, N_SAMPLES // 128, 128).transpose(0, 2, 1, 3)
  ids2d = ids.reshape(NBLOCKS, 128)
  parts = _sc_partials(data4, ids)
  y2tc, cnttc = _tc_main(data4, ids2d)
  return _tc_finalize(parts, y2tc.reshape(-1),
                      cnttc.reshape(-1)).reshape(NUM_TYPES, N_PROPS)


# speedup vs baseline: 6.1687x; 4.7281x over previous
import functools

import jax
import jax.numpy as jnp
from jax import lax
from jax.experimental import pallas as pl
from jax.experimental.pallas import tpu as pltpu
from jax.experimental.pallas import tpu_sc as plsc

NUM_TYPES = 100
N_SAMPLES = 3_200_000
N_PROPS = 16

NW = 32
NBLK = 25
CHUNK = NBLK * 128
NBLOCKS = N_SAMPLES // 128

SC_BLOCKS = 19350
NCHUNKS_MAIN = SC_BLOCKS // NBLK
NCHUNKS = NCHUNKS_MAIN + 1
SLOTS = -(-NCHUNKS // NW)
TC_G = 75
TC_OFF = SC_BLOCKS // TC_G
TC_STEPS = (NBLOCKS - 25 - SC_BLOCKS) // TC_G
TAIL_CHUNK = NBLOCKS // NBLK - 1
FIX_SLOTS = -(-TC_STEPS // NW)


def _sc_partials(data4, ids):
  mesh = plsc.VectorSubcoreMesh(core_axis_name="c", subcore_axis_name="s")

  @functools.partial(
      pl.kernel,
      out_type=jax.ShapeDtypeStruct((NW * 2 * NUM_TYPES * N_PROPS,),
                                    jnp.float32),
      mesh=mesh,
      compiler_params=pltpu.CompilerParams(
          needs_layout_passes=False, use_tc_tiling_on_sc=False),
      scratch_types=[
          pltpu.VMEM((NBLK, 8, 128), jnp.float32),
          pltpu.VMEM((NBLK, 8, 128), jnp.float32),
          pltpu.VMEM((NBLK, 8, 128), jnp.float32),
          pltpu.VMEM((NBLK, 8, 128), jnp.float32),
          pltpu.VMEM((16,), jnp.int32),
          pltpu.VMEM((16,), jnp.int32),
          pltpu.VMEM((16,), jnp.int32),
          pltpu.VMEM((16,), jnp.int32),
          pltpu.VMEM((CHUNK,), jnp.int32),
          pltpu.VMEM((NUM_TYPES * N_PROPS,), jnp.float32),
          pltpu.VMEM((NUM_TYPES * N_PROPS,), jnp.float32),
          pltpu.VMEM((256,), jnp.float32),
          pltpu.SemaphoreType.DMA,
          pltpu.SemaphoreType.DMA,
          pltpu.SemaphoreType.DMA,
          pltpu.SemaphoreType.DMA,
      ],
  )
  def k(data_hbm, ids_hbm, out_hbm, lo0, hi0, lo1, hi1, idsF0, idsL0,
        idsF1, idsL1, idsfull, acc, cnt, tt, sd0, sd1, si0, si1):
    wid = lax.axis_index("c") * 16 + lax.axis_index("s")
    nc = jnp.where(wid < NCHUNKS - (SLOTS - 1) * NW, SLOTS, SLOTS - 1)
    iota16 = lax.iota(jnp.int32, 16)
    iota16x16 = iota16 * 16
    zeros16 = jnp.zeros((16,), jnp.float32)
    ones16 = jnp.ones((16,), jnp.float32)

    def zbody(kk, _):
      acc[pl.ds(kk * 16, 16)] = zeros16
      cnt[pl.ds(kk * 16, 16)] = zeros16
      return 0
    lax.fori_loop(0, NUM_TYPES, zbody, 0)

    def chunk_of(slot):
      c_lin = wid + jnp.minimum(slot, nc - 1) * NW
      return jnp.where(c_lin >= NCHUNKS_MAIN, TAIL_CHUNK, c_lin)

    def start(c, lo, hi, idsF, idsL, sd, si):
      b = c * NBLK
      pltpu.make_async_copy(data_hbm.at[0, pl.ds(b, NBLK)], lo, sd).start()
      pltpu.make_async_copy(data_hbm.at[1, pl.ds(b, NBLK)], hi, sd).start()
      pltpu.make_async_copy(ids_hbm.at[pl.ds(c * CHUNK, 16)], idsF,
                            si).start()
      pltpu.make_async_copy(ids_hbm.at[pl.ds(c * CHUNK + CHUNK - 16, 16)],
                            idsL, si).start()

    def wait(c, lo, hi, idsF, idsL, sd, si):
      b = c * NBLK
      pltpu.make_async_copy(data_hbm.at[0, pl.ds(b, NBLK)], lo, sd).wait()
      pltpu.make_async_copy(data_hbm.at[1, pl.ds(b, NBLK)], hi, sd).wait()
      pltpu.make_async_copy(ids_hbm.at[pl.ds(c * CHUNK, 16)], idsF,
                            si).wait()
      pltpu.make_async_copy(ids_hbm.at[pl.ds(c * CHUNK + CHUNK - 16, 16)],
                            idsL, si).wait()

    def lanesum_from_tt():
      tot = plsc.load_gather(tt, [iota16x16])
      for l in range(1, 16):
        tot = tot + plsc.load_gather(tt, [iota16x16 + l])
      return tot

    def flush_accp(accp, seg, n_samples):
      for p in range(16):
        tt[pl.ds(p * 16, 16)] = accp[p]
      tot = lanesum_from_tt()
      idx = jnp.full((16,), seg * 16, jnp.int32) + iota16
      plsc.addupdate_scatter(acc, [idx], tot)
      plsc.addupdate_scatter(cnt, [idx],
                             jnp.full((16,), n_samples, jnp.float32))

    def accum_block(lo, hi, blk, accp):
      out = list(accp)
      for half, buf in ((0, lo), (1, hi)):
        for j in range(8):
          p = half * 8 + j
          a = out[p]
          for kk in range(8):
            v = buf[blk, j, pl.ds(kk * 16, 16)]
            a = a + v * v
          out[p] = a
      return tuple(out)

    def process(c, lo, hi, idsF, idsL):
      first = idsF[...][0]
      last = idsL[...][15]
      uniform = first == last

      @pl.when(uniform)
      def _fast():
        accp = lax.fori_loop(
            0, NBLK, lambda blk, accs: accum_block(lo, hi, blk, accs),
            tuple(zeros16 for _ in range(16)))
        flush_accp(accp, first, float(CHUNK))

      @pl.when(jnp.logical_not(uniform))
      def _slow():
        pltpu.sync_copy(ids_hbm.at[pl.ds(c * CHUNK, CHUNK)], idsfull)

        def blk_body(blk, _):
          boff = blk * 128
          bfirst = idsfull[pl.ds(boff, 16)][0]
          blast = idsfull[pl.ds(boff + 112, 16)][15]

          @pl.when(bfirst == blast)
          def _ublock():
            accp = accum_block(lo, hi, blk, tuple(zeros16 for _ in range(16)))
            flush_accp(accp, bfirst, 128.0)

          @pl.when(jnp.logical_not(bfirst == blast))
          def _bblock():
            def sub_body(kk, _):
              segs = idsfull[pl.ds(boff + kk * 16, 16)]
              for half, buf in ((0, lo), (1, hi)):
                for j in range(8):
                  tt[pl.ds((half * 8 + j) * 16, 16)] = (
                      buf[blk, j, pl.ds(kk * 16, 16)])
              for l in range(16):
                col = plsc.load_gather(tt, [iota16x16 + l])
                idx = jnp.full((16,), segs[l] * 16, jnp.int32) + iota16
                plsc.addupdate_scatter(acc, [idx], col * col)
                plsc.addupdate_scatter(cnt, [idx], ones16)
              return 0
            lax.fori_loop(0, 8, sub_body, 0)
          return 0
        lax.fori_loop(0, NBLK, blk_body, 0)

    start(chunk_of(0), lo0, hi0, idsF0, idsL0, sd0, si0)
    start(chunk_of(1), lo1, hi1, idsF1, idsL1, sd1, si1)

    def outer(kk, _):
      n0 = 2 * kk

      @pl.when(n0 < nc)
      def _w0():
        c = chunk_of(n0)
        wait(c, lo0, hi0, idsF0, idsL0, sd0, si0)
        process(c, lo0, hi0, idsF0, idsL0)

      @pl.when(n0 + 2 < nc)
      def _s0():
        start(chunk_of(n0 + 2), lo0, hi0, idsF0, idsL0, sd0, si0)

      @pl.when(n0 + 1 < nc)
      def _w1():
        c = chunk_of(n0 + 1)
        wait(c, lo1, hi1, idsF1, idsL1, sd1, si1)
        process(c, lo1, hi1, idsF1, idsL1)

      @pl.when(n0 + 3 < nc)
      def _s1():
        start(chunk_of(n0 + 3), lo1, hi1, idsF1, idsL1, sd1, si1)
      return 0
    lax.fori_loop(0, (SLOTS + 1) // 2, outer, 0)

    def fix_body(j, _):
      t = wid + j * NW

      @pl.when(t < TC_STEPS)
      def _fix():
        s0 = (SC_BLOCKS + t * TC_G) * 128
        pltpu.sync_copy(ids_hbm.at[pl.ds(s0, 16)], idsF0)
        pltpu.sync_copy(ids_hbm.at[pl.ds(s0 + TC_G * 128 - 16, 16)], idsL0)
        tfirst = idsF0[...][0]
        tlast = idsL0[...][15]

        @pl.when(tfirst != tlast)
        def _dofix():
          cb = (SC_BLOCKS + t * TC_G) // NBLK

          def piece_body(p, _):
            c = cb + p
            start(c, lo0, hi0, idsF0, idsL0, sd0, si0)
            wait(c, lo0, hi0, idsF0, idsL0, sd0, si0)
            process(c, lo0, hi0, idsF0, idsL0)
            return 0
          lax.fori_loop(0, 3, piece_body, 0)
      return 0
    lax.fori_loop(0, FIX_SLOTS, fix_body, 0)

    base = wid * 2 * NUM_TYPES * N_PROPS
    pltpu.sync_copy(acc, out_hbm.at[pl.ds(base, NUM_TYPES * N_PROPS)])
    pltpu.sync_copy(
        cnt, out_hbm.at[pl.ds(base + NUM_TYPES * N_PROPS,
                              NUM_TYPES * N_PROPS)])

  return k(data4, ids)


def _tc_main(data4, firsts, lasts):
  def body(lo_ref, hi_ref, f_ref, l_ref, y2_ref, cnt_ref,
           acc0, acc1, y2acc, cntacc, rseg, rn):
    i = pl.program_id(0)
    cmin = f_ref[i]
    cmax = l_ref[i]
    uniform = cmin == cmax
    z8 = jnp.zeros((8, 128), jnp.float32)

    @pl.when(i == 0)
    def _init():
      y2acc[...] = jnp.zeros((NUM_TYPES, N_PROPS), jnp.float32)
      cntacc[...] = jnp.zeros((NUM_TYPES, N_PROPS), jnp.float32)
      acc0[...] = z8
      acc1[...] = z8
      rseg[0] = cmin
      rn[0] = 0.0

    def flush():
      rs = rseg[0]
      row = jnp.concatenate(
          [jnp.sum(acc0[...], axis=1), jnp.sum(acc1[...], axis=1)])[None, :]
      y2acc[pl.ds(rs, 1), :] = y2acc[pl.ds(rs, 1), :] + row
      cntacc[pl.ds(rs, 1), :] = cntacc[pl.ds(rs, 1), :] + rn[0]
      acc0[...] = z8
      acc1[...] = z8
      rn[0] = 0.0

    @pl.when(uniform)
    def _u():
      @pl.when(cmin != rseg[0])
      def _sw():
        flush()
        rseg[0] = cmin
      a0 = acc0[...]
      a1 = acc1[...]
      for g in range(TC_G):
        v0 = lo_ref[0, g]
        v1 = hi_ref[0, g]
        a0 = a0 + v0 * v0
        a1 = a1 + v1 * v1
      acc0[...] = a0
      acc1[...] = a1
      rn[0] = rn[0] + float(TC_G * 128)

    @pl.when(jnp.logical_not(uniform))
    def _b():
      flush()
      rseg[0] = cmax

    @pl.when(i == TC_STEPS - 1)
    def _fin():
      flush()
      y2_ref[...] = y2acc[...]
      cnt_ref[...] = cntacc[...]

  return pl.pallas_call(
      body,
      grid=(TC_STEPS,),
      in_specs=[
          pl.BlockSpec((1, TC_G, 8, 128), lambda i: (0, TC_OFF + i, 0, 0)),
          pl.BlockSpec((1, TC_G, 8, 128), lambda i: (1, TC_OFF + i, 0, 0)),
          pl.BlockSpec(memory_space=pltpu.SMEM),
          pl.BlockSpec(memory_space=pltpu.SMEM),
      ],
      out_specs=[
          pl.BlockSpec((NUM_TYPES, N_PROPS), lambda i: (0, 0)),
          pl.BlockSpec((NUM_TYPES, N_PROPS), lambda i: (0, 0)),
      ],
      out_shape=[
          jax.ShapeDtypeStruct((NUM_TYPES, N_PROPS), jnp.float32),
          jax.ShapeDtypeStruct((NUM_TYPES, N_PROPS), jnp.float32),
      ],
      scratch_shapes=[
          pltpu.VMEM((8, 128), jnp.float32),
          pltpu.VMEM((8, 128), jnp.float32),
          pltpu.VMEM((NUM_TYPES, N_PROPS), jnp.float32),
          pltpu.VMEM((NUM_TYPES, N_PROPS), jnp.float32),
          pltpu.SMEM((1,), jnp.int32),
          pltpu.SMEM((1,), jnp.float32),
      ],
  )(data4, data4, firsts, lasts)


def _tc_finalize(parts, y2tc, cnttc):
  d = NUM_TYPES * N_PROPS

  def body(p_ref, ytc_ref, ctc_ref, o_ref):
    y2 = ytc_ref[...]
    c = ctc_ref[...]
    for w in range(NW):
      y2 = y2 + p_ref[pl.ds(w * 2 * d, d)]
      c = c + p_ref[pl.ds(w * 2 * d + d, d)]
    o_ref[...] = jnp.where(c > 0.0, jnp.sqrt(y2 / jnp.maximum(c, 1.0)),
                           jnp.float32(1.0))

  return pl.pallas_call(
      body,
      out_shape=jax.ShapeDtypeStruct((d,), jnp.float32),
  )(parts, y2tc, cnttc)


@jax.jit
def kernel(data, segment_ids):
  ids = segment_ids.astype(jnp.int32)
  data4 = data.T.reshape(2, 8, N_SAMPLES // 128, 128).transpose(0, 2, 1, 3)
  base = SC_BLOCKS * 128
  step = TC_G * 128
  end = base + TC_STEPS * step
  firsts = lax.slice(ids, (base,), (end,), (step,))
  lasts = lax.slice(ids, (base + step - 1,), (end,), (step,))
  parts = _sc_partials(data4, ids)
  y2tc, cnttc = _tc_main(data4, firsts, lasts)
  return _tc_finalize(parts, y2tc.reshape(-1),
                      cnttc.reshape(-1)).reshape(NUM_TYPES, N_PROPS)


# speedup vs baseline: 6.6946x; 1.0853x over previous
import functools

import jax
import jax.numpy as jnp
from jax import lax
from jax.experimental import pallas as pl
from jax.experimental.pallas import tpu as pltpu
from jax.experimental.pallas import tpu_sc as plsc

NUM_TYPES = 100
N_SAMPLES = 3_200_000
N_PROPS = 16

NW = 32
NBLK = 25
CHUNK = NBLK * 128
NBLOCKS = N_SAMPLES // 128

SC_BLOCKS = 20000
NCHUNKS_MAIN = SC_BLOCKS // NBLK
NCHUNKS = NCHUNKS_MAIN
SLOTS = -(-NCHUNKS // NW)
TC_G = 25
TC_OFF = SC_BLOCKS // TC_G
TC_STEPS = (NBLOCKS - SC_BLOCKS) // TC_G
TAIL_CHUNK = NCHUNKS_MAIN
FIX_SLOTS = -(-TC_STEPS // NW)


def _sc_partials(data4, ids):
  mesh = plsc.VectorSubcoreMesh(core_axis_name="c", subcore_axis_name="s")

  @functools.partial(
      pl.kernel,
      out_type=jax.ShapeDtypeStruct((NW * 2 * NUM_TYPES * N_PROPS,),
                                    jnp.float32),
      mesh=mesh,
      compiler_params=pltpu.CompilerParams(
          needs_layout_passes=False, use_tc_tiling_on_sc=False),
      scratch_types=[
          pltpu.VMEM((NBLK, 8, 128), jnp.float32),
          pltpu.VMEM((NBLK, 8, 128), jnp.float32),
          pltpu.VMEM((NBLK, 8, 128), jnp.float32),
          pltpu.VMEM((NBLK, 8, 128), jnp.float32),
          pltpu.VMEM((16,), jnp.int32),
          pltpu.VMEM((16,), jnp.int32),
          pltpu.VMEM((16,), jnp.int32),
          pltpu.VMEM((16,), jnp.int32),
          pltpu.VMEM((CHUNK,), jnp.int32),
          pltpu.VMEM((NUM_TYPES * N_PROPS,), jnp.float32),
          pltpu.VMEM((NUM_TYPES * N_PROPS,), jnp.float32),
          pltpu.VMEM((256,), jnp.float32),
          pltpu.SemaphoreType.DMA,
          pltpu.SemaphoreType.DMA,
          pltpu.SemaphoreType.DMA,
          pltpu.SemaphoreType.DMA,
      ],
  )
  def k(data_hbm, ids_hbm, out_hbm, lo0, hi0, lo1, hi1, idsF0, idsL0,
        idsF1, idsL1, idsfull, acc, cnt, tt, sd0, sd1, si0, si1):
    wid = lax.axis_index("c") * 16 + lax.axis_index("s")
    nc = jnp.where(wid < NCHUNKS - (SLOTS - 1) * NW, SLOTS, SLOTS - 1)
    iota16 = lax.iota(jnp.int32, 16)
    iota16x16 = iota16 * 16
    zeros16 = jnp.zeros((16,), jnp.float32)
    ones16 = jnp.ones((16,), jnp.float32)

    def zbody(kk, _):
      acc[pl.ds(kk * 16, 16)] = zeros16
      cnt[pl.ds(kk * 16, 16)] = zeros16
      return 0
    lax.fori_loop(0, NUM_TYPES, zbody, 0)

    def chunk_of(slot):
      c_lin = wid + jnp.minimum(slot, nc - 1) * NW
      return jnp.where(c_lin >= NCHUNKS_MAIN, TAIL_CHUNK, c_lin)

    def start(c, lo, hi, idsF, idsL, sd, si):
      b = c * NBLK
      pltpu.make_async_copy(data_hbm.at[0, pl.ds(b, NBLK)], lo, sd).start()
      pltpu.make_async_copy(data_hbm.at[1, pl.ds(b, NBLK)], hi, sd).start()
      pltpu.make_async_copy(ids_hbm.at[pl.ds(c * CHUNK, 16)], idsF,
                            si).start()
      pltpu.make_async_copy(ids_hbm.at[pl.ds(c * CHUNK + CHUNK - 16, 16)],
                            idsL, si).start()

    def wait(c, lo, hi, idsF, idsL, sd, si):
      b = c * NBLK
      pltpu.make_async_copy(data_hbm.at[0, pl.ds(b, NBLK)], lo, sd).wait()
      pltpu.make_async_copy(data_hbm.at[1, pl.ds(b, NBLK)], hi, sd).wait()
      pltpu.make_async_copy(ids_hbm.at[pl.ds(c * CHUNK, 16)], idsF,
                            si).wait()
      pltpu.make_async_copy(ids_hbm.at[pl.ds(c * CHUNK + CHUNK - 16, 16)],
                            idsL, si).wait()

    def lanesum_from_tt():
      tot = plsc.load_gather(tt, [iota16x16])
      for l in range(1, 16):
        tot = tot + plsc.load_gather(tt, [iota16x16 + l])
      return tot

    def flush_accp(accp, seg, n_samples):
      for p in range(16):
        tt[pl.ds(p * 16, 16)] = accp[p]
      tot = lanesum_from_tt()
      idx = jnp.full((16,), seg * 16, jnp.int32) + iota16
      plsc.addupdate_scatter(acc, [idx], tot)
      plsc.addupdate_scatter(cnt, [idx],
                             jnp.full((16,), n_samples, jnp.float32))

    def accum_block(lo, hi, blk, accp):
      out = list(accp)
      for half, buf in ((0, lo), (1, hi)):
        for j in range(8):
          p = half * 8 + j
          a = out[p]
          for kk in range(8):
            v = buf[blk, j, pl.ds(kk * 16, 16)]
            a = a + v * v
          out[p] = a
      return tuple(out)

    def process(c, lo, hi, idsF, idsL):
      first = idsF[...][0]
      last = idsL[...][15]
      uniform = first == last

      @pl.when(uniform)
      def _fast():
        accp = lax.fori_loop(
            0, NBLK, lambda blk, accs: accum_block(lo, hi, blk, accs),
            tuple(zeros16 for _ in range(16)))
        flush_accp(accp, first, float(CHUNK))

      @pl.when(jnp.logical_not(uniform))
      def _slow():
        pltpu.sync_copy(ids_hbm.at[pl.ds(c * CHUNK, CHUNK)], idsfull)

        def blk_body(blk, _):
          boff = blk * 128
          bfirst = idsfull[pl.ds(boff, 16)][0]
          blast = idsfull[pl.ds(boff + 112, 16)][15]

          @pl.when(bfirst == blast)
          def _ublock():
            accp = accum_block(lo, hi, blk, tuple(zeros16 for _ in range(16)))
            flush_accp(accp, bfirst, 128.0)

          @pl.when(jnp.logical_not(bfirst == blast))
          def _bblock():
            def sub_body(kk, _):
              segs = idsfull[pl.ds(boff + kk * 16, 16)]
              for half, buf in ((0, lo), (1, hi)):
                for j in range(8):
                  tt[pl.ds((half * 8 + j) * 16, 16)] = (
                      buf[blk, j, pl.ds(kk * 16, 16)])
              for l in range(16):
                col = plsc.load_gather(tt, [iota16x16 + l])
                idx = jnp.full((16,), segs[l] * 16, jnp.int32) + iota16
                plsc.addupdate_scatter(acc, [idx], col * col)
                plsc.addupdate_scatter(cnt, [idx], ones16)
              return 0
            lax.fori_loop(0, 8, sub_body, 0)
          return 0
        lax.fori_loop(0, NBLK, blk_body, 0)

    start(chunk_of(0), lo0, hi0, idsF0, idsL0, sd0, si0)
    start(chunk_of(1), lo1, hi1, idsF1, idsL1, sd1, si1)

    def outer(kk, _):
      n0 = 2 * kk

      @pl.when(n0 < nc)
      def _w0():
        c = chunk_of(n0)
        wait(c, lo0, hi0, idsF0, idsL0, sd0, si0)
        process(c, lo0, hi0, idsF0, idsL0)

      @pl.when(n0 + 2 < nc)
      def _s0():
        start(chunk_of(n0 + 2), lo0, hi0, idsF0, idsL0, sd0, si0)

      @pl.when(n0 + 1 < nc)
      def _w1():
        c = chunk_of(n0 + 1)
        wait(c, lo1, hi1, idsF1, idsL1, sd1, si1)
        process(c, lo1, hi1, idsF1, idsL1)

      @pl.when(n0 + 3 < nc)
      def _s1():
        start(chunk_of(n0 + 3), lo1, hi1, idsF1, idsL1, sd1, si1)
      return 0
    lax.fori_loop(0, (SLOTS + 1) // 2, outer, 0)

    def fix_body(j, _):
      t = wid + j * NW

      @pl.when(t < TC_STEPS)
      def _fix():
        s0 = (SC_BLOCKS + t * TC_G) * 128
        pltpu.sync_copy(ids_hbm.at[pl.ds(s0, 16)], idsF0)
        pltpu.sync_copy(ids_hbm.at[pl.ds(s0 + TC_G * 128 - 16, 16)], idsL0)
        tfirst = idsF0[...][0]
        tlast = idsL0[...][15]

        @pl.when(tfirst != tlast)
        def _dofix():
          c = (SC_BLOCKS + t * TC_G) // NBLK
          start(c, lo0, hi0, idsF0, idsL0, sd0, si0)
          wait(c, lo0, hi0, idsF0, idsL0, sd0, si0)
          process(c, lo0, hi0, idsF0, idsL0)
      return 0
    lax.fori_loop(0, FIX_SLOTS, fix_body, 0)

    base = wid * 2 * NUM_TYPES * N_PROPS
    pltpu.sync_copy(acc, out_hbm.at[pl.ds(base, NUM_TYPES * N_PROPS)])
    pltpu.sync_copy(
        cnt, out_hbm.at[pl.ds(base + NUM_TYPES * N_PROPS,
                              NUM_TYPES * N_PROPS)])

  return k(data4, ids)


def _tc_main(data4, firsts, lasts):
  def body(lo_ref, hi_ref, f_ref, l_ref, y2_ref, cnt_ref,
           acc0, acc1, y2acc, cntacc, rseg, rn):
    i = pl.program_id(0)
    cmin = f_ref[i]
    cmax = l_ref[i]
    uniform = cmin == cmax
    z8 = jnp.zeros((8, 128), jnp.float32)

    @pl.when(i == 0)
    def _init():
      y2acc[...] = jnp.zeros((NUM_TYPES, N_PROPS), jnp.float32)
      cntacc[...] = jnp.zeros((NUM_TYPES, N_PROPS), jnp.float32)
      acc0[...] = z8
      acc1[...] = z8
      rseg[0] = cmin
      rn[0] = 0.0

    def flush():
      rs = rseg[0]
      row = jnp.concatenate(
          [jnp.sum(acc0[...], axis=1), jnp.sum(acc1[...], axis=1)])[None, :]
      y2acc[pl.ds(rs, 1), :] = y2acc[pl.ds(rs, 1), :] + row
      cntacc[pl.ds(rs, 1), :] = cntacc[pl.ds(rs, 1), :] + rn[0]
      acc0[...] = z8
      acc1[...] = z8
      rn[0] = 0.0

    @pl.when(uniform)
    def _u():
      @pl.when(cmin != rseg[0])
      def _sw():
        flush()
        rseg[0] = cmin
      a0 = acc0[...]
      a1 = acc1[...]
      for g in range(TC_G):
        v0 = lo_ref[0, g]
        v1 = hi_ref[0, g]
        a0 = a0 + v0 * v0
        a1 = a1 + v1 * v1
      acc0[...] = a0
      acc1[...] = a1
      rn[0] = rn[0] + float(TC_G * 128)

    @pl.when(jnp.logical_not(uniform))
    def _b():
      flush()
      rseg[0] = cmax

    @pl.when(i == TC_STEPS - 1)
    def _fin():
      flush()
      y2_ref[...] = y2acc[...]
      cnt_ref[...] = cntacc[...]

  return pl.pallas_call(
      body,
      grid=(TC_STEPS,),
      in_specs=[
          pl.BlockSpec((1, TC_G, 8, 128), lambda i: (0, TC_OFF + i, 0, 0)),
          pl.BlockSpec((1, TC_G, 8, 128), lambda i: (1, TC_OFF + i, 0, 0)),
          pl.BlockSpec(memory_space=pltpu.SMEM),
          pl.BlockSpec(memory_space=pltpu.SMEM),
      ],
      out_specs=[
          pl.BlockSpec((NUM_TYPES, N_PROPS), lambda i: (0, 0)),
          pl.BlockSpec((NUM_TYPES, N_PROPS), lambda i: (0, 0)),
      ],
      out_shape=[
          jax.ShapeDtypeStruct((NUM_TYPES, N_PROPS), jnp.float32),
          jax.ShapeDtypeStruct((NUM_TYPES, N_PROPS), jnp.float32),
      ],
      scratch_shapes=[
          pltpu.VMEM((8, 128), jnp.float32),
          pltpu.VMEM((8, 128), jnp.float32),
          pltpu.VMEM((NUM_TYPES, N_PROPS), jnp.float32),
          pltpu.VMEM((NUM_TYPES, N_PROPS), jnp.float32),
          pltpu.SMEM((1,), jnp.int32),
          pltpu.SMEM((1,), jnp.float32),
      ],
  )(data4, data4, firsts, lasts)


def _tc_finalize(parts, y2tc, cnttc):
  d = NUM_TYPES * N_PROPS

  def body(p_ref, ytc_ref, ctc_ref, o_ref):
    y2 = ytc_ref[...]
    c = ctc_ref[...]
    for w in range(NW):
      y2 = y2 + p_ref[pl.ds(w * 2 * d, d)]
      c = c + p_ref[pl.ds(w * 2 * d + d, d)]
    o_ref[...] = jnp.where(c > 0.0, jnp.sqrt(y2 / jnp.maximum(c, 1.0)),
                           jnp.float32(1.0))

  return pl.pallas_call(
      body,
      out_shape=jax.ShapeDtypeStruct((d,), jnp.float32),
  )(parts, y2tc, cnttc)


@jax.jit
def kernel(data, segment_ids):
  ids = segment_ids.astype(jnp.int32)
  data4 = data.T.reshape(2, 8, N_SAMPLES // 128, 128).transpose(0, 2, 1, 3)
  base = SC_BLOCKS * 128
  step = TC_G * 128
  end = base + TC_STEPS * step
  firsts = lax.slice(ids, (base,), (end,), (step,))
  lasts = lax.slice(ids, (base + step - 1,), (end,), (step,))
  parts = _sc_partials(data4, ids)
  y2tc, cnttc = _tc_main(data4, firsts, lasts)
  return _tc_finalize(parts, y2tc.reshape(-1),
                      cnttc.reshape(-1)).reshape(NUM_TYPES, N_PROPS)


# speedup vs baseline: 6.6987x; 1.0006x over previous
import functools

import jax
import jax.numpy as jnp
from jax import lax
from jax.experimental import pallas as pl
from jax.experimental.pallas import tpu as pltpu
from jax.experimental.pallas import tpu_sc as plsc

NUM_TYPES = 100
N_SAMPLES = 3_200_000
N_PROPS = 16

NW = 32
NBLK = 25
CHUNK = NBLK * 128
NBLOCKS = N_SAMPLES // 128

SC_BLOCKS = 20000
NCHUNKS_MAIN = SC_BLOCKS // NBLK
NCHUNKS = NCHUNKS_MAIN
SLOTS = -(-NCHUNKS // NW)
TC_G = 25
TC_OFF = SC_BLOCKS // TC_G
TC_STEPS = (NBLOCKS - SC_BLOCKS) // TC_G
TAIL_CHUNK = NCHUNKS_MAIN
FIX_SLOTS = -(-TC_STEPS // NW)


def _sc_partials(data4, ids):
  mesh = plsc.VectorSubcoreMesh(core_axis_name="c", subcore_axis_name="s")

  @functools.partial(
      pl.kernel,
      out_type=jax.ShapeDtypeStruct((NW * 2 * NUM_TYPES * N_PROPS,),
                                    jnp.float32),
      mesh=mesh,
      compiler_params=pltpu.CompilerParams(
          needs_layout_passes=False, use_tc_tiling_on_sc=False),
      scratch_types=[
          pltpu.VMEM((NBLK, 8, 128), jnp.float32),
          pltpu.VMEM((NBLK, 8, 128), jnp.float32),
          pltpu.VMEM((NBLK, 8, 128), jnp.float32),
          pltpu.VMEM((NBLK, 8, 128), jnp.float32),
          pltpu.VMEM((16,), jnp.int32),
          pltpu.VMEM((16,), jnp.int32),
          pltpu.VMEM((16,), jnp.int32),
          pltpu.VMEM((16,), jnp.int32),
          pltpu.VMEM((CHUNK,), jnp.int32),
          pltpu.VMEM((NUM_TYPES * N_PROPS,), jnp.float32),
          pltpu.VMEM((NUM_TYPES * N_PROPS,), jnp.float32),
          pltpu.VMEM((256,), jnp.float32),
          pltpu.VMEM((16,), jnp.int32),
          pltpu.VMEM((16,), jnp.int32),
          pltpu.VMEM((16,), jnp.int32),
          pltpu.SMEM((16,), jnp.int32),
          pltpu.SemaphoreType.DMA,
          pltpu.SemaphoreType.DMA,
          pltpu.SemaphoreType.DMA,
          pltpu.SemaphoreType.DMA,
      ],
  )
  def k(data_hbm, ids_hbm, out_hbm, lo0, hi0, lo1, hi1, idsF0, idsL0,
        idsF1, idsL1, idsfull, acc, cnt, tt, gidx, ghead, gtail, gflag,
        sd0, sd1, si0, si1):
    wid = lax.axis_index("c") * 16 + lax.axis_index("s")
    nc = jnp.where(wid < NCHUNKS - (SLOTS - 1) * NW, SLOTS, SLOTS - 1)
    iota16 = lax.iota(jnp.int32, 16)
    iota16x16 = iota16 * 16
    zeros16 = jnp.zeros((16,), jnp.float32)
    ones16 = jnp.ones((16,), jnp.float32)

    def zbody(kk, _):
      acc[pl.ds(kk * 16, 16)] = zeros16
      cnt[pl.ds(kk * 16, 16)] = zeros16
      return 0
    lax.fori_loop(0, NUM_TYPES, zbody, 0)

    def chunk_of(slot):
      c_lin = wid + jnp.minimum(slot, nc - 1) * NW
      return jnp.where(c_lin >= NCHUNKS_MAIN, TAIL_CHUNK, c_lin)

    def start(c, lo, hi, idsF, idsL, sd, si):
      b = c * NBLK
      pltpu.make_async_copy(data_hbm.at[0, pl.ds(b, NBLK)], lo, sd).start()
      pltpu.make_async_copy(data_hbm.at[1, pl.ds(b, NBLK)], hi, sd).start()
      pltpu.make_async_copy(ids_hbm.at[pl.ds(c * CHUNK, 16)], idsF,
                            si).start()
      pltpu.make_async_copy(ids_hbm.at[pl.ds(c * CHUNK + CHUNK - 16, 16)],
                            idsL, si).start()

    def wait(c, lo, hi, idsF, idsL, sd, si):
      b = c * NBLK
      pltpu.make_async_copy(data_hbm.at[0, pl.ds(b, NBLK)], lo, sd).wait()
      pltpu.make_async_copy(data_hbm.at[1, pl.ds(b, NBLK)], hi, sd).wait()
      pltpu.make_async_copy(ids_hbm.at[pl.ds(c * CHUNK, 16)], idsF,
                            si).wait()
      pltpu.make_async_copy(ids_hbm.at[pl.ds(c * CHUNK + CHUNK - 16, 16)],
                            idsL, si).wait()

    def lanesum_from_tt():
      tot = plsc.load_gather(tt, [iota16x16])
      for l in range(1, 16):
        tot = tot + plsc.load_gather(tt, [iota16x16 + l])
      return tot

    def flush_accp(accp, seg, n_samples):
      for p in range(16):
        tt[pl.ds(p * 16, 16)] = accp[p]
      tot = lanesum_from_tt()
      idx = jnp.full((16,), seg * 16, jnp.int32) + iota16
      plsc.addupdate_scatter(acc, [idx], tot)
      plsc.addupdate_scatter(cnt, [idx],
                             jnp.full((16,), n_samples, jnp.float32))

    def accum_block(lo, hi, blk, accp):
      out = list(accp)
      for half, buf in ((0, lo), (1, hi)):
        for j in range(8):
          p = half * 8 + j
          a = out[p]
          for kk in range(8):
            v = buf[blk, j, pl.ds(kk * 16, 16)]
            a = a + v * v
          out[p] = a
      return tuple(out)

    def process(c, lo, hi, idsF, idsL):
      first = idsF[...][0]
      last = idsL[...][15]
      uniform = first == last

      @pl.when(uniform)
      def _fast():
        accp = lax.fori_loop(
            0, NBLK, lambda blk, accs: accum_block(lo, hi, blk, accs),
            tuple(zeros16 for _ in range(16)))
        flush_accp(accp, first, float(CHUNK))

      @pl.when(jnp.logical_not(uniform))
      def _slow():
        pltpu.sync_copy(ids_hbm.at[pl.ds(c * CHUNK, CHUNK)], idsfull)

        def blk_body(blk, _):
          boff = blk * 128
          bfirst = idsfull[pl.ds(boff, 16)][0]
          blast = idsfull[pl.ds(boff + 112, 16)][15]

          @pl.when(bfirst == blast)
          def _ublock():
            accp = accum_block(lo, hi, blk, tuple(zeros16 for _ in range(16)))
            flush_accp(accp, bfirst, 128.0)

          @pl.when(jnp.logical_not(bfirst == blast))
          def _bblock():
            def sub_body(kk, _):
              segs = idsfull[pl.ds(boff + kk * 16, 16)]
              for half, buf in ((0, lo), (1, hi)):
                for j in range(8):
                  tt[pl.ds((half * 8 + j) * 16, 16)] = (
                      buf[blk, j, pl.ds(kk * 16, 16)])
              for l in range(16):
                col = plsc.load_gather(tt, [iota16x16 + l])
                idx = jnp.full((16,), segs[l] * 16, jnp.int32) + iota16
                plsc.addupdate_scatter(acc, [idx], col * col)
                plsc.addupdate_scatter(cnt, [idx], ones16)
              return 0
            lax.fori_loop(0, 8, sub_body, 0)
          return 0
        lax.fori_loop(0, NBLK, blk_body, 0)

    start(chunk_of(0), lo0, hi0, idsF0, idsL0, sd0, si0)
    start(chunk_of(1), lo1, hi1, idsF1, idsL1, sd1, si1)

    def outer(kk, _):
      n0 = 2 * kk

      @pl.when(n0 < nc)
      def _w0():
        c = chunk_of(n0)
        wait(c, lo0, hi0, idsF0, idsL0, sd0, si0)
        process(c, lo0, hi0, idsF0, idsL0)

      @pl.when(n0 + 2 < nc)
      def _s0():
        start(chunk_of(n0 + 2), lo0, hi0, idsF0, idsL0, sd0, si0)

      @pl.when(n0 + 1 < nc)
      def _w1():
        c = chunk_of(n0 + 1)
        wait(c, lo1, hi1, idsF1, idsL1, sd1, si1)
        process(c, lo1, hi1, idsF1, idsL1)

      @pl.when(n0 + 3 < nc)
      def _s1():
        start(chunk_of(n0 + 3), lo1, hi1, idsF1, idsL1, sd1, si1)
      return 0
    lax.fori_loop(0, (SLOTS + 1) // 2, outer, 0)

    hbase = SC_BLOCKS * 128 + wid * CHUNK
    hidx = jnp.minimum(jnp.full((16,), hbase, jnp.int32) + iota16 * (NW * CHUNK),
                       N_SAMPLES - 1)
    gidx[...] = hidx
    pltpu.async_copy(ids_hbm.at[gidx], ghead, sd0).wait()
    gidx[...] = jnp.minimum(hidx + (CHUNK - 1), N_SAMPLES - 1)
    pltpu.async_copy(ids_hbm.at[gidx], gtail, sd0).wait()
    heads = ghead[...]
    tails = gtail[...]
    for j in range(FIX_SLOTS):
      t_ok = (wid + j * NW) < TC_STEPS
      gflag[j] = jnp.where(
          jnp.logical_and(t_ok, heads[j] != tails[j]), 1, 0)

    def fix_body(j, _):
      @pl.when(gflag[j] == 1)
      def _dofix():
        c = NCHUNKS_MAIN + wid + j * NW
        start(c, lo0, hi0, idsF0, idsL0, sd0, si0)
        wait(c, lo0, hi0, idsF0, idsL0, sd0, si0)
        process(c, lo0, hi0, idsF0, idsL0)
      return 0
    lax.fori_loop(0, FIX_SLOTS, fix_body, 0)

    base = wid * 2 * NUM_TYPES * N_PROPS
    pltpu.sync_copy(acc, out_hbm.at[pl.ds(base, NUM_TYPES * N_PROPS)])
    pltpu.sync_copy(
        cnt, out_hbm.at[pl.ds(base + NUM_TYPES * N_PROPS,
                              NUM_TYPES * N_PROPS)])

  return k(data4, ids)


def _tc_main(data4, firsts, lasts):
  def body(lo_ref, hi_ref, f_ref, l_ref, y2_ref, cnt_ref,
           acc0, acc1, y2acc, cntacc, rseg, rn):
    i = pl.program_id(0)
    cmin = f_ref[i]
    cmax = l_ref[i]
    uniform = cmin == cmax
    z8 = jnp.zeros((8, 128), jnp.float32)

    @pl.when(i == 0)
    def _init():
      y2acc[...] = jnp.zeros((NUM_TYPES, N_PROPS), jnp.float32)
      cntacc[...] = jnp.zeros((NUM_TYPES, N_PROPS), jnp.float32)
      acc0[...] = z8
      acc1[...] = z8
      rseg[0] = cmin
      rn[0] = 0.0

    def flush():
      rs = rseg[0]
      row = jnp.concatenate(
          [jnp.sum(acc0[...], axis=1), jnp.sum(acc1[...], axis=1)])[None, :]
      y2acc[pl.ds(rs, 1), :] = y2acc[pl.ds(rs, 1), :] + row
      cntacc[pl.ds(rs, 1), :] = cntacc[pl.ds(rs, 1), :] + rn[0]
      acc0[...] = z8
      acc1[...] = z8
      rn[0] = 0.0

    @pl.when(uniform)
    def _u():
      @pl.when(cmin != rseg[0])
      def _sw():
        flush()
        rseg[0] = cmin
      a0 = acc0[...]
      a1 = acc1[...]
      for g in range(TC_G):
        v0 = lo_ref[0, g]
        v1 = hi_ref[0, g]
        a0 = a0 + v0 * v0
        a1 = a1 + v1 * v1
      acc0[...] = a0
      acc1[...] = a1
      rn[0] = rn[0] + float(TC_G * 128)

    @pl.when(jnp.logical_not(uniform))
    def _b():
      flush()
      rseg[0] = cmax

    @pl.when(i == TC_STEPS - 1)
    def _fin():
      flush()
      y2_ref[...] = y2acc[...]
      cnt_ref[...] = cntacc[...]

  return pl.pallas_call(
      body,
      grid=(TC_STEPS,),
      in_specs=[
          pl.BlockSpec((1, TC_G, 8, 128), lambda i: (0, TC_OFF + i, 0, 0)),
          pl.BlockSpec((1, TC_G, 8, 128), lambda i: (1, TC_OFF + i, 0, 0)),
          pl.BlockSpec(memory_space=pltpu.SMEM),
          pl.BlockSpec(memory_space=pltpu.SMEM),
      ],
      out_specs=[
          pl.BlockSpec((NUM_TYPES, N_PROPS), lambda i: (0, 0)),
          pl.BlockSpec((NUM_TYPES, N_PROPS), lambda i: (0, 0)),
      ],
      out_shape=[
          jax.ShapeDtypeStruct((NUM_TYPES, N_PROPS), jnp.float32),
          jax.ShapeDtypeStruct((NUM_TYPES, N_PROPS), jnp.float32),
      ],
      scratch_shapes=[
          pltpu.VMEM((8, 128), jnp.float32),
          pltpu.VMEM((8, 128), jnp.float32),
          pltpu.VMEM((NUM_TYPES, N_PROPS), jnp.float32),
          pltpu.VMEM((NUM_TYPES, N_PROPS), jnp.float32),
          pltpu.SMEM((1,), jnp.int32),
          pltpu.SMEM((1,), jnp.float32),
      ],
  )(data4, data4, firsts, lasts)


def _tc_finalize(parts, y2tc, cnttc):
  d = NUM_TYPES * N_PROPS

  def body(p_ref, ytc_ref, ctc_ref, o_ref):
    y2 = ytc_ref[...]
    c = ctc_ref[...]
    for w in range(NW):
      y2 = y2 + p_ref[pl.ds(w * 2 * d, d)]
      c = c + p_ref[pl.ds(w * 2 * d + d, d)]
    o_ref[...] = jnp.where(c > 0.0, jnp.sqrt(y2 / jnp.maximum(c, 1.0)),
                           jnp.float32(1.0))

  return pl.pallas_call(
      body,
      out_shape=jax.ShapeDtypeStruct((d,), jnp.float32),
  )(parts, y2tc, cnttc)


@jax.jit
def kernel(data, segment_ids):
  ids = segment_ids.astype(jnp.int32)
  data4 = data.T.reshape(2, 8, N_SAMPLES // 128, 128).transpose(0, 2, 1, 3)
  base = SC_BLOCKS * 128
  step = TC_G * 128
  end = base + TC_STEPS * step
  firsts = lax.slice(ids, (base,), (end,), (step,))
  lasts = lax.slice(ids, (base + step - 1,), (end,), (step,))
  parts = _sc_partials(data4, ids)
  y2tc, cnttc = _tc_main(data4, firsts, lasts)
  return _tc_finalize(parts, y2tc.reshape(-1),
                      cnttc.reshape(-1)).reshape(NUM_TYPES, N_PROPS)


# speedup vs baseline: 7.2427x; 1.0812x over previous
import functools

import jax
import jax.numpy as jnp
from jax import lax
from jax.experimental import pallas as pl
from jax.experimental.pallas import tpu as pltpu
from jax.experimental.pallas import tpu_sc as plsc

NUM_TYPES = 100
N_SAMPLES = 3_200_000
N_PROPS = 16

NW = 32
NBLK = 20
CHUNK = NBLK * 128
NBLOCKS = N_SAMPLES // 128

SC_BLOCKS = 19600
NCHUNKS_MAIN = SC_BLOCKS // NBLK
NCHUNKS = NCHUNKS_MAIN
SLOTS = -(-NCHUNKS // NW)
TC_G = 40
TC_OFF = SC_BLOCKS // TC_G
TC_STEPS = (NBLOCKS - SC_BLOCKS) // TC_G
TAIL_CHUNK = NCHUNKS_MAIN
FIX_SLOTS = -(-TC_STEPS // NW)


def _sc_partials(data4, ids):
  mesh = plsc.VectorSubcoreMesh(core_axis_name="c", subcore_axis_name="s")

  @functools.partial(
      pl.kernel,
      out_type=jax.ShapeDtypeStruct((NW * 2 * NUM_TYPES * N_PROPS,),
                                    jnp.float32),
      mesh=mesh,
      compiler_params=pltpu.CompilerParams(
          needs_layout_passes=False, use_tc_tiling_on_sc=False),
      scratch_types=[
          pltpu.VMEM((NBLK, 8, 128), jnp.float32),
          pltpu.VMEM((NBLK, 8, 128), jnp.float32),
          pltpu.VMEM((NBLK, 8, 128), jnp.float32),
          pltpu.VMEM((NBLK, 8, 128), jnp.float32),
          pltpu.VMEM((16,), jnp.int32),
          pltpu.VMEM((16,), jnp.int32),
          pltpu.VMEM((16,), jnp.int32),
          pltpu.VMEM((16,), jnp.int32),
          pltpu.VMEM((CHUNK,), jnp.int32),
          pltpu.VMEM((NUM_TYPES * N_PROPS,), jnp.float32),
          pltpu.VMEM((NUM_TYPES * N_PROPS,), jnp.float32),
          pltpu.VMEM((256,), jnp.float32),
          pltpu.VMEM((16,), jnp.int32),
          pltpu.VMEM((16,), jnp.int32),
          pltpu.VMEM((16,), jnp.int32),
          pltpu.SMEM((16,), jnp.int32),
          pltpu.SemaphoreType.DMA,
          pltpu.SemaphoreType.DMA,
          pltpu.SemaphoreType.DMA,
          pltpu.SemaphoreType.DMA,
      ],
  )
  def k(data_hbm, ids_hbm, out_hbm, lo0, hi0, lo1, hi1, idsF0, idsL0,
        idsF1, idsL1, idsfull, acc, cnt, tt, gidx, ghead, gtail, gflag,
        sd0, sd1, si0, si1):
    wid = lax.axis_index("c") * 16 + lax.axis_index("s")
    nc = jnp.where(wid < NCHUNKS - (SLOTS - 1) * NW, SLOTS, SLOTS - 1)
    iota16 = lax.iota(jnp.int32, 16)
    iota16x16 = iota16 * 16
    zeros16 = jnp.zeros((16,), jnp.float32)
    ones16 = jnp.ones((16,), jnp.float32)

    def zbody(kk, _):
      acc[pl.ds(kk * 16, 16)] = zeros16
      cnt[pl.ds(kk * 16, 16)] = zeros16
      return 0
    lax.fori_loop(0, NUM_TYPES, zbody, 0)

    def chunk_of(slot):
      c_lin = wid + jnp.minimum(slot, nc - 1) * NW
      return jnp.where(c_lin >= NCHUNKS_MAIN, TAIL_CHUNK, c_lin)

    def start(c, lo, hi, idsF, idsL, sd, si):
      b = c * NBLK
      pltpu.make_async_copy(data_hbm.at[0, pl.ds(b, NBLK)], lo, sd).start()
      pltpu.make_async_copy(data_hbm.at[1, pl.ds(b, NBLK)], hi, sd).start()
      pltpu.make_async_copy(ids_hbm.at[pl.ds(c * CHUNK, 16)], idsF,
                            si).start()
      pltpu.make_async_copy(ids_hbm.at[pl.ds(c * CHUNK + CHUNK - 16, 16)],
                            idsL, si).start()

    def wait(c, lo, hi, idsF, idsL, sd, si):
      b = c * NBLK
      pltpu.make_async_copy(data_hbm.at[0, pl.ds(b, NBLK)], lo, sd).wait()
      pltpu.make_async_copy(data_hbm.at[1, pl.ds(b, NBLK)], hi, sd).wait()
      pltpu.make_async_copy(ids_hbm.at[pl.ds(c * CHUNK, 16)], idsF,
                            si).wait()
      pltpu.make_async_copy(ids_hbm.at[pl.ds(c * CHUNK + CHUNK - 16, 16)],
                            idsL, si).wait()

    def lanesum_from_tt():
      tot = plsc.load_gather(tt, [iota16x16])
      for l in range(1, 16):
        tot = tot + plsc.load_gather(tt, [iota16x16 + l])
      return tot

    def flush_accp(accp, seg, n_samples):
      for p in range(16):
        tt[pl.ds(p * 16, 16)] = accp[p]
      tot = lanesum_from_tt()
      idx = jnp.full((16,), seg * 16, jnp.int32) + iota16
      plsc.addupdate_scatter(acc, [idx], tot)
      plsc.addupdate_scatter(cnt, [idx],
                             jnp.full((16,), n_samples, jnp.float32))

    def accum_block(lo, hi, blk, accp):
      out = list(accp)
      for half, buf in ((0, lo), (1, hi)):
        for j in range(8):
          p = half * 8 + j
          a = out[p]
          for kk in range(8):
            v = buf[blk, j, pl.ds(kk * 16, 16)]
            a = a + v * v
          out[p] = a
      return tuple(out)

    def process(c, lo, hi, idsF, idsL):
      first = idsF[...][0]
      last = idsL[...][15]
      uniform = first == last

      @pl.when(uniform)
      def _fast():
        accp = lax.fori_loop(
            0, NBLK, lambda blk, accs: accum_block(lo, hi, blk, accs),
            tuple(zeros16 for _ in range(16)))
        flush_accp(accp, first, float(CHUNK))

      @pl.when(jnp.logical_not(uniform))
      def _slow():
        pltpu.sync_copy(ids_hbm.at[pl.ds(c * CHUNK, CHUNK)], idsfull)

        def blk_body(blk, _):
          boff = blk * 128
          bfirst = idsfull[pl.ds(boff, 16)][0]
          blast = idsfull[pl.ds(boff + 112, 16)][15]

          @pl.when(bfirst == blast)
          def _ublock():
            accp = accum_block(lo, hi, blk, tuple(zeros16 for _ in range(16)))
            flush_accp(accp, bfirst, 128.0)

          @pl.when(jnp.logical_not(bfirst == blast))
          def _bblock():
            def sub_body(kk, _):
              segs = idsfull[pl.ds(boff + kk * 16, 16)]
              for half, buf in ((0, lo), (1, hi)):
                for j in range(8):
                  tt[pl.ds((half * 8 + j) * 16, 16)] = (
                      buf[blk, j, pl.ds(kk * 16, 16)])
              for l in range(16):
                col = plsc.load_gather(tt, [iota16x16 + l])
                idx = jnp.full((16,), segs[l] * 16, jnp.int32) + iota16
                plsc.addupdate_scatter(acc, [idx], col * col)
                plsc.addupdate_scatter(cnt, [idx], ones16)
              return 0
            lax.fori_loop(0, 8, sub_body, 0)
          return 0
        lax.fori_loop(0, NBLK, blk_body, 0)

    start(chunk_of(0), lo0, hi0, idsF0, idsL0, sd0, si0)
    start(chunk_of(1), lo1, hi1, idsF1, idsL1, sd1, si1)

    def outer(kk, _):
      n0 = 2 * kk

      @pl.when(n0 < nc)
      def _w0():
        c = chunk_of(n0)
        wait(c, lo0, hi0, idsF0, idsL0, sd0, si0)
        process(c, lo0, hi0, idsF0, idsL0)

      @pl.when(n0 + 2 < nc)
      def _s0():
        start(chunk_of(n0 + 2), lo0, hi0, idsF0, idsL0, sd0, si0)

      @pl.when(n0 + 1 < nc)
      def _w1():
        c = chunk_of(n0 + 1)
        wait(c, lo1, hi1, idsF1, idsL1, sd1, si1)
        process(c, lo1, hi1, idsF1, idsL1)

      @pl.when(n0 + 3 < nc)
      def _s1():
        start(chunk_of(n0 + 3), lo1, hi1, idsF1, idsL1, sd1, si1)
      return 0
    lax.fori_loop(0, (SLOTS + 1) // 2, outer, 0)

    tcchunk = TC_G * 128
    hbase = SC_BLOCKS * 128 + wid * tcchunk
    hidx = jnp.minimum(
        jnp.full((16,), hbase, jnp.int32) + iota16 * (NW * tcchunk),
        N_SAMPLES - 1)
    gidx[...] = hidx
    pltpu.async_copy(ids_hbm.at[gidx], ghead, sd0).wait()
    gidx[...] = jnp.minimum(hidx + (tcchunk - 1), N_SAMPLES - 1)
    pltpu.async_copy(ids_hbm.at[gidx], gtail, sd0).wait()
    heads = ghead[...]
    tails = gtail[...]
    for j in range(FIX_SLOTS):
      t_ok = (wid + j * NW) < TC_STEPS
      gflag[j] = jnp.where(
          jnp.logical_and(t_ok, heads[j] != tails[j]), 1, 0)

    def fix_body(j, _):
      @pl.when(gflag[j] == 1)
      def _dofix():
        c0 = NCHUNKS_MAIN + 2 * (wid + j * NW)

        def piece_body(p, _):
          c = c0 + p
          start(c, lo0, hi0, idsF0, idsL0, sd0, si0)
          wait(c, lo0, hi0, idsF0, idsL0, sd0, si0)
          process(c, lo0, hi0, idsF0, idsL0)
          return 0
        lax.fori_loop(0, 2, piece_body, 0)
      return 0
    lax.fori_loop(0, FIX_SLOTS, fix_body, 0)

    base = wid * 2 * NUM_TYPES * N_PROPS
    pltpu.sync_copy(acc, out_hbm.at[pl.ds(base, NUM_TYPES * N_PROPS)])
    pltpu.sync_copy(
        cnt, out_hbm.at[pl.ds(base + NUM_TYPES * N_PROPS,
                              NUM_TYPES * N_PROPS)])

  return k(data4, ids)


def _tc_main(data4, ids2d):
  def body(lo_ref, hi_ref, ids_ref, y2_ref, cnt_ref,
           acc0, acc1, y2acc, cntacc, rseg, rn):
    i = pl.program_id(0)
    ids_blk = ids_ref[...]
    cmin = jnp.min(ids_blk)
    cmax = jnp.max(ids_blk)
    uniform = cmin == cmax
    z8 = jnp.zeros((8, 128), jnp.float32)

    @pl.when(i == 0)
    def _init():
      y2acc[...] = jnp.zeros((NUM_TYPES, N_PROPS), jnp.float32)
      cntacc[...] = jnp.zeros((NUM_TYPES, N_PROPS), jnp.float32)
      acc0[...] = z8
      acc1[...] = z8
      rseg[0] = cmin
      rn[0] = 0.0

    def flush():
      rs = rseg[0]
      row = jnp.concatenate(
          [jnp.sum(acc0[...], axis=1), jnp.sum(acc1[...], axis=1)])[None, :]
      y2acc[pl.ds(rs, 1), :] = y2acc[pl.ds(rs, 1), :] + row
      cntacc[pl.ds(rs, 1), :] = cntacc[pl.ds(rs, 1), :] + rn[0]
      acc0[...] = z8
      acc1[...] = z8
      rn[0] = 0.0

    @pl.when(uniform)
    def _u():
      @pl.when(cmin != rseg[0])
      def _sw():
        flush()
        rseg[0] = cmin
      a0 = acc0[...]
      a1 = acc1[...]
      for g in range(TC_G):
        v0 = lo_ref[0, g]
        v1 = hi_ref[0, g]
        a0 = a0 + v0 * v0
        a1 = a1 + v1 * v1
      acc0[...] = a0
      acc1[...] = a1
      rn[0] = rn[0] + float(TC_G * 128)

    @pl.when(jnp.logical_not(uniform))
    def _b():
      flush()
      rseg[0] = cmax

    @pl.when(i == TC_STEPS - 1)
    def _fin():
      flush()
      y2_ref[...] = y2acc[...]
      cnt_ref[...] = cntacc[...]

  return pl.pallas_call(
      body,
      grid=(TC_STEPS,),
      in_specs=[
          pl.BlockSpec((1, TC_G, 8, 128), lambda i: (0, TC_OFF + i, 0, 0)),
          pl.BlockSpec((1, TC_G, 8, 128), lambda i: (1, TC_OFF + i, 0, 0)),
          pl.BlockSpec((TC_G, 128), lambda i: (TC_OFF + i, 0)),
      ],
      out_specs=[
          pl.BlockSpec((NUM_TYPES, N_PROPS), lambda i: (0, 0)),
          pl.BlockSpec((NUM_TYPES, N_PROPS), lambda i: (0, 0)),
      ],
      out_shape=[
          jax.ShapeDtypeStruct((NUM_TYPES, N_PROPS), jnp.float32),
          jax.ShapeDtypeStruct((NUM_TYPES, N_PROPS), jnp.float32),
      ],
      scratch_shapes=[
          pltpu.VMEM((8, 128), jnp.float32),
          pltpu.VMEM((8, 128), jnp.float32),
          pltpu.VMEM((NUM_TYPES, N_PROPS), jnp.float32),
          pltpu.VMEM((NUM_TYPES, N_PROPS), jnp.float32),
          pltpu.SMEM((1,), jnp.int32),
          pltpu.SMEM((1,), jnp.float32),
      ],
  )(data4, data4, ids2d)


def _tc_finalize(parts, y2tc, cnttc):
  d = NUM_TYPES * N_PROPS

  def body(p_ref, ytc_ref, ctc_ref, o_ref):
    y2 = ytc_ref[...]
    c = ctc_ref[...]
    for w in range(NW):
      y2 = y2 + p_ref[pl.ds(w * 2 * d, d)]
      c = c + p_ref[pl.ds(w * 2 * d + d, d)]
    o_ref[...] = jnp.where(c > 0.0, jnp.sqrt(y2 / jnp.maximum(c, 1.0)),
                           jnp.float32(1.0))

  return pl.pallas_call(
      body,
      out_shape=jax.ShapeDtypeStruct((d,), jnp.float32),
  )(parts, y2tc, cnttc)


@jax.jit
def kernel(data, segment_ids):
  ids = segment_ids.astype(jnp.int32)
  data4 = data.T.reshape(2, 8, N_SAMPLES // 128, 128).transpose(0, 2, 1, 3)
  ids2d = ids.reshape(NBLOCKS, 128)
  parts = _sc_partials(data4, ids)
  y2tc, cnttc = _tc_main(data4, ids2d)
  return _tc_finalize(parts, y2tc.reshape(-1),
                      cnttc.reshape(-1)).reshape(NUM_TYPES, N_PROPS)


# speedup vs baseline: 7.2813x; 1.0053x over previous
import functools

import jax
import jax.numpy as jnp
from jax import lax
from jax.experimental import pallas as pl
from jax.experimental.pallas import tpu as pltpu
from jax.experimental.pallas import tpu_sc as plsc

NUM_TYPES = 100
N_SAMPLES = 3_200_000
N_PROPS = 16

NW = 32
NBLK = 20
CHUNK = NBLK * 128
NBLOCKS = N_SAMPLES // 128

SC_BLOCKS = 19000
NCHUNKS_MAIN = SC_BLOCKS // NBLK
NCHUNKS = NCHUNKS_MAIN
SLOTS = -(-NCHUNKS // NW)
TC_G = 40
TC_OFF = SC_BLOCKS // TC_G
TC_STEPS = (NBLOCKS - SC_BLOCKS) // TC_G
TAIL_CHUNK = NCHUNKS_MAIN
FIX_SLOTS = -(-TC_STEPS // NW)


def _sc_partials(data4, ids):
  mesh = plsc.VectorSubcoreMesh(core_axis_name="c", subcore_axis_name="s")

  @functools.partial(
      pl.kernel,
      out_type=jax.ShapeDtypeStruct((NW * 2 * NUM_TYPES * N_PROPS,),
                                    jnp.float32),
      mesh=mesh,
      compiler_params=pltpu.CompilerParams(
          needs_layout_passes=False, use_tc_tiling_on_sc=False),
      scratch_types=[
          pltpu.VMEM((NBLK, 8, 128), jnp.float32),
          pltpu.VMEM((NBLK, 8, 128), jnp.float32),
          pltpu.VMEM((NBLK, 8, 128), jnp.float32),
          pltpu.VMEM((NBLK, 8, 128), jnp.float32),
          pltpu.VMEM((16,), jnp.int32),
          pltpu.VMEM((16,), jnp.int32),
          pltpu.VMEM((16,), jnp.int32),
          pltpu.VMEM((16,), jnp.int32),
          pltpu.VMEM((CHUNK,), jnp.int32),
          pltpu.VMEM((NUM_TYPES * N_PROPS,), jnp.float32),
          pltpu.VMEM((NUM_TYPES * N_PROPS,), jnp.float32),
          pltpu.VMEM((256,), jnp.float32),
          pltpu.VMEM((16,), jnp.int32),
          pltpu.VMEM((16,), jnp.int32),
          pltpu.VMEM((16,), jnp.int32),
          pltpu.SMEM((16,), jnp.int32),
          pltpu.SemaphoreType.DMA,
          pltpu.SemaphoreType.DMA,
          pltpu.SemaphoreType.DMA,
          pltpu.SemaphoreType.DMA,
      ],
  )
  def k(data_hbm, ids_hbm, out_hbm, lo0, hi0, lo1, hi1, idsF0, idsL0,
        idsF1, idsL1, idsfull, acc, cnt, tt, gidx, ghead, gtail, gflag,
        sd0, sd1, si0, si1):
    wid = lax.axis_index("c") * 16 + lax.axis_index("s")
    nc = jnp.where(wid < NCHUNKS - (SLOTS - 1) * NW, SLOTS, SLOTS - 1)
    iota16 = lax.iota(jnp.int32, 16)
    iota16x16 = iota16 * 16
    zeros16 = jnp.zeros((16,), jnp.float32)
    ones16 = jnp.ones((16,), jnp.float32)

    def zbody(kk, _):
      acc[pl.ds(kk * 16, 16)] = zeros16
      cnt[pl.ds(kk * 16, 16)] = zeros16
      return 0
    lax.fori_loop(0, NUM_TYPES, zbody, 0)

    def chunk_of(slot):
      c_lin = wid + jnp.minimum(slot, nc - 1) * NW
      return jnp.where(c_lin >= NCHUNKS_MAIN, TAIL_CHUNK, c_lin)

    def start(c, lo, hi, idsF, idsL, sd, si):
      b = c * NBLK
      pltpu.make_async_copy(data_hbm.at[0, pl.ds(b, NBLK)], lo, sd).start()
      pltpu.make_async_copy(data_hbm.at[1, pl.ds(b, NBLK)], hi, sd).start()
      pltpu.make_async_copy(ids_hbm.at[pl.ds(c * CHUNK, 16)], idsF,
                            si).start()
      pltpu.make_async_copy(ids_hbm.at[pl.ds(c * CHUNK + CHUNK - 16, 16)],
                            idsL, si).start()

    def wait(c, lo, hi, idsF, idsL, sd, si):
      b = c * NBLK
      pltpu.make_async_copy(data_hbm.at[0, pl.ds(b, NBLK)], lo, sd).wait()
      pltpu.make_async_copy(data_hbm.at[1, pl.ds(b, NBLK)], hi, sd).wait()
      pltpu.make_async_copy(ids_hbm.at[pl.ds(c * CHUNK, 16)], idsF,
                            si).wait()
      pltpu.make_async_copy(ids_hbm.at[pl.ds(c * CHUNK + CHUNK - 16, 16)],
                            idsL, si).wait()

    def lanesum_from_tt():
      tot = plsc.load_gather(tt, [iota16x16])
      for l in range(1, 16):
        tot = tot + plsc.load_gather(tt, [iota16x16 + l])
      return tot

    def flush_accp(accp, seg, n_samples):
      for p in range(16):
        tt[pl.ds(p * 16, 16)] = accp[p]
      tot = lanesum_from_tt()
      idx = jnp.full((16,), seg * 16, jnp.int32) + iota16
      plsc.addupdate_scatter(acc, [idx], tot)
      plsc.addupdate_scatter(cnt, [idx],
                             jnp.full((16,), n_samples, jnp.float32))

    def accum_block(lo, hi, blk, accp):
      out = list(accp)
      for half, buf in ((0, lo), (1, hi)):
        for j in range(8):
          p = half * 8 + j
          a = out[p]
          for kk in range(8):
            v = buf[blk, j, pl.ds(kk * 16, 16)]
            a = a + v * v
          out[p] = a
      return tuple(out)

    def process(c, lo, hi, idsF, idsL):
      first = idsF[...][0]
      last = idsL[...][15]
      uniform = first == last

      @pl.when(uniform)
      def _fast():
        accp = lax.fori_loop(
            0, NBLK, lambda blk, accs: accum_block(lo, hi, blk, accs),
            tuple(zeros16 for _ in range(16)))
        flush_accp(accp, first, float(CHUNK))

      @pl.when(jnp.logical_not(uniform))
      def _slow():
        pltpu.sync_copy(ids_hbm.at[pl.ds(c * CHUNK, CHUNK)], idsfull)

        def blk_body(blk, _):
          boff = blk * 128
          bfirst = idsfull[pl.ds(boff, 16)][0]
          blast = idsfull[pl.ds(boff + 112, 16)][15]

          @pl.when(bfirst == blast)
          def _ublock():
            accp = accum_block(lo, hi, blk, tuple(zeros16 for _ in range(16)))
            flush_accp(accp, bfirst, 128.0)

          @pl.when(jnp.logical_not(bfirst == blast))
          def _bblock():
            def sub_body(kk, _):
              segs = idsfull[pl.ds(boff + kk * 16, 16)]
              for half, buf in ((0, lo), (1, hi)):
                for j in range(8):
                  tt[pl.ds((half * 8 + j) * 16, 16)] = (
                      buf[blk, j, pl.ds(kk * 16, 16)])
              for l in range(16):
                col = plsc.load_gather(tt, [iota16x16 + l])
                idx = jnp.full((16,), segs[l] * 16, jnp.int32) + iota16
                plsc.addupdate_scatter(acc, [idx], col * col)
                plsc.addupdate_scatter(cnt, [idx], ones16)
              return 0
            lax.fori_loop(0, 8, sub_body, 0)
          return 0
        lax.fori_loop(0, NBLK, blk_body, 0)

    start(chunk_of(0), lo0, hi0, idsF0, idsL0, sd0, si0)
    start(chunk_of(1), lo1, hi1, idsF1, idsL1, sd1, si1)

    def outer(kk, _):
      n0 = 2 * kk

      @pl.when(n0 < nc)
      def _w0():
        c = chunk_of(n0)
        wait(c, lo0, hi0, idsF0, idsL0, sd0, si0)
        process(c, lo0, hi0, idsF0, idsL0)

      @pl.when(n0 + 2 < nc)
      def _s0():
        start(chunk_of(n0 + 2), lo0, hi0, idsF0, idsL0, sd0, si0)

      @pl.when(n0 + 1 < nc)
      def _w1():
        c = chunk_of(n0 + 1)
        wait(c, lo1, hi1, idsF1, idsL1, sd1, si1)
        process(c, lo1, hi1, idsF1, idsL1)

      @pl.when(n0 + 3 < nc)
      def _s1():
        start(chunk_of(n0 + 3), lo1, hi1, idsF1, idsL1, sd1, si1)
      return 0
    lax.fori_loop(0, (SLOTS + 1) // 2, outer, 0)

    tcchunk = TC_G * 128
    hbase = SC_BLOCKS * 128 + wid * tcchunk
    hidx = jnp.minimum(
        jnp.full((16,), hbase, jnp.int32) + iota16 * (NW * tcchunk),
        N_SAMPLES - 1)
    gidx[...] = hidx
    pltpu.async_copy(ids_hbm.at[gidx], ghead, sd0).wait()
    gidx[...] = jnp.minimum(hidx + (tcchunk - 1), N_SAMPLES - 1)
    pltpu.async_copy(ids_hbm.at[gidx], gtail, sd0).wait()
    heads = ghead[...]
    tails = gtail[...]
    for j in range(FIX_SLOTS):
      t_ok = (wid + j * NW) < TC_STEPS
      gflag[j] = jnp.where(
          jnp.logical_and(t_ok, heads[j] != tails[j]), 1, 0)

    def fix_body(j, _):
      @pl.when(gflag[j] == 1)
      def _dofix():
        c0 = NCHUNKS_MAIN + 2 * (wid + j * NW)

        def piece_body(p, _):
          c = c0 + p
          start(c, lo0, hi0, idsF0, idsL0, sd0, si0)
          wait(c, lo0, hi0, idsF0, idsL0, sd0, si0)
          process(c, lo0, hi0, idsF0, idsL0)
          return 0
        lax.fori_loop(0, 2, piece_body, 0)
      return 0
    lax.fori_loop(0, FIX_SLOTS, fix_body, 0)

    base = wid * 2 * NUM_TYPES * N_PROPS
    pltpu.sync_copy(acc, out_hbm.at[pl.ds(base, NUM_TYPES * N_PROPS)])
    pltpu.sync_copy(
        cnt, out_hbm.at[pl.ds(base + NUM_TYPES * N_PROPS,
                              NUM_TYPES * N_PROPS)])

  return k(data4, ids)


def _tc_main(data4, ids2d):
  def body(lo_ref, hi_ref, ids_ref, y2_ref, cnt_ref,
           acc0, acc1, y2acc, cntacc, rseg, rn):
    i = pl.program_id(0)
    ids_blk = ids_ref[...]
    cmin = jnp.min(ids_blk)
    cmax = jnp.max(ids_blk)
    uniform = cmin == cmax
    z8 = jnp.zeros((8, 128), jnp.float32)

    @pl.when(i == 0)
    def _init():
      y2acc[...] = jnp.zeros((NUM_TYPES, N_PROPS), jnp.float32)
      cntacc[...] = jnp.zeros((NUM_TYPES, N_PROPS), jnp.float32)
      acc0[...] = z8
      acc1[...] = z8
      rseg[0] = cmin
      rn[0] = 0.0

    def flush():
      rs = rseg[0]
      row = jnp.concatenate(
          [jnp.sum(acc0[...], axis=1), jnp.sum(acc1[...], axis=1)])[None, :]
      y2acc[pl.ds(rs, 1), :] = y2acc[pl.ds(rs, 1), :] + row
      cntacc[pl.ds(rs, 1), :] = cntacc[pl.ds(rs, 1), :] + rn[0]
      acc0[...] = z8
      acc1[...] = z8
      rn[0] = 0.0

    @pl.when(uniform)
    def _u():
      @pl.when(cmin != rseg[0])
      def _sw():
        flush()
        rseg[0] = cmin
      a0 = acc0[...]
      a1 = acc1[...]
      for g in range(TC_G):
        v0 = lo_ref[0, g]
        v1 = hi_ref[0, g]
        a0 = a0 + v0 * v0
        a1 = a1 + v1 * v1
      acc0[...] = a0
      acc1[...] = a1
      rn[0] = rn[0] + float(TC_G * 128)

    @pl.when(jnp.logical_not(uniform))
    def _b():
      flush()
      rseg[0] = cmax

    @pl.when(i == TC_STEPS - 1)
    def _fin():
      flush()
      y2_ref[...] = y2acc[...]
      cnt_ref[...] = cntacc[...]

  return pl.pallas_call(
      body,
      grid=(TC_STEPS,),
      in_specs=[
          pl.BlockSpec((1, TC_G, 8, 128), lambda i: (0, TC_OFF + i, 0, 0)),
          pl.BlockSpec((1, TC_G, 8, 128), lambda i: (1, TC_OFF + i, 0, 0)),
          pl.BlockSpec((TC_G, 128), lambda i: (TC_OFF + i, 0)),
      ],
      out_specs=[
          pl.BlockSpec((NUM_TYPES, N_PROPS), lambda i: (0, 0)),
          pl.BlockSpec((NUM_TYPES, N_PROPS), lambda i: (0, 0)),
      ],
      out_shape=[
          jax.ShapeDtypeStruct((NUM_TYPES, N_PROPS), jnp.float32),
          jax.ShapeDtypeStruct((NUM_TYPES, N_PROPS), jnp.float32),
      ],
      scratch_shapes=[
          pltpu.VMEM((8, 128), jnp.float32),
          pltpu.VMEM((8, 128), jnp.float32),
          pltpu.VMEM((NUM_TYPES, N_PROPS), jnp.float32),
          pltpu.VMEM((NUM_TYPES, N_PROPS), jnp.float32),
          pltpu.SMEM((1,), jnp.int32),
          pltpu.SMEM((1,), jnp.float32),
      ],
  )(data4, data4, ids2d)


def _tc_finalize(parts, y2tc, cnttc):
  d = NUM_TYPES * N_PROPS

  def body(p_ref, ytc_ref, ctc_ref, o_ref):
    y2 = ytc_ref[...]
    c = ctc_ref[...]
    for w in range(NW):
      y2 = y2 + p_ref[pl.ds(w * 2 * d, d)]
      c = c + p_ref[pl.ds(w * 2 * d + d, d)]
    o_ref[...] = jnp.where(c > 0.0, jnp.sqrt(y2 / jnp.maximum(c, 1.0)),
                           jnp.float32(1.0))

  return pl.pallas_call(
      body,
      out_shape=jax.ShapeDtypeStruct((d,), jnp.float32),
  )(parts, y2tc, cnttc)


@jax.jit
def kernel(data, segment_ids):
  ids = segment_ids.astype(jnp.int32)
  data4 = data.T.reshape(2, 8, N_SAMPLES // 128, 128).transpose(0, 2, 1, 3)
  ids2d = ids.reshape(NBLOCKS, 128)
  parts = _sc_partials(data4, ids)
  y2tc, cnttc = _tc_main(data4, ids2d)
  return _tc_finalize(parts, y2tc.reshape(-1),
                      cnttc.reshape(-1)).reshape(NUM_TYPES, N_PROPS)
